# baseline (speedup 1.0000x reference)
import functools

import jax
import jax.numpy as jnp
from jax import lax
from jax.experimental import pallas as pl
from jax.experimental.pallas import tpu as pltpu
from jax.experimental.pallas import tpu_sc as plsc

NSEG = 2048
NC = 2
NS = 16
NW = NC * NS
LANES = 16
CHUNK = 2048


def _pass1_call(src, tgt, idx):
  n = idx.shape[0]
  ppt = n // NW
  nsub = ppt // CHUNK
  ngrp = CHUNK // LANES
  mesh = plsc.VectorSubcoreMesh(core_axis_name="c", subcore_axis_name="s")

  @functools.partial(
      pl.kernel, mesh=mesh,
      compiler_params=pltpu.CompilerParams(needs_layout_passes=False, use_tc_tiling_on_sc=False),
      out_type=jax.ShapeDtypeStruct((NW, 16, NSEG), jnp.float32),
      scratch_types=[
          pltpu.VMEM((CHUNK, 3), jnp.float32),
          pltpu.VMEM((CHUNK, 3), jnp.float32),
          pltpu.VMEM((CHUNK,), jnp.int32),
          pltpu.VMEM((16, NSEG), jnp.float32),
      ],
  )
  def k(src_h, tgt_h, idx_h, out_h, sv, tv, iv, tab):
    wid = lax.axis_index("s") * NC + lax.axis_index("c")
    iota = lax.iota(jnp.int32, LANES)
    c0 = jnp.zeros((LANES,), jnp.int32)
    c1 = jnp.full((LANES,), 1, jnp.int32)
    c2 = jnp.full((LANES,), 2, jnp.int32)
    ones = jnp.ones((LANES,), jnp.float32)
    jrows = [jnp.full((LANES,), j, jnp.int32) for j in range(16)]

    zeros16 = jnp.zeros((LANES,), jnp.float32)

    def zero_body(i, carry):
      for j in range(16):
        tab[j, pl.ds(i * LANES, LANES)] = zeros16
      return carry
    lax.fori_loop(0, NSEG // LANES, zero_body, 0)

    def sub_body(sub, carry):
      base = wid * ppt + sub * CHUNK
      pltpu.sync_copy(src_h.at[pl.ds(base, CHUNK)], sv)
      pltpu.sync_copy(tgt_h.at[pl.ds(base, CHUNK)], tv)
      pltpu.sync_copy(idx_h.at[pl.ds(base, CHUNK)], iv)

      def grp_body(g, c):
        rows = g * LANES + iota
        ivec = iv[pl.ds(g * LANES, LANES)]
        sx = plsc.load_gather(sv, [rows, c0])
        sy = plsc.load_gather(sv, [rows, c1])
        sz = plsc.load_gather(sv, [rows, c2])
        tx = plsc.load_gather(tv, [rows, c0])
        ty = plsc.load_gather(tv, [rows, c1])
        tz = plsc.load_gather(tv, [rows, c2])
        vals = (ones, sx, sy, sz, tx, ty, tz,
                sx * tx, sx * ty, sx * tz,
                sy * tx, sy * ty, sy * tz,
                sz * tx, sz * ty, sz * tz)
        for j, v in enumerate(vals):
          plsc.addupdate_scatter(tab, [jrows[j], ivec], v)
        return c
      lax.fori_loop(0, ngrp, grp_body, 0)
      return carry
    lax.fori_loop(0, nsub, sub_body, 0)
    pltpu.sync_copy(tab, out_h.at[wid])

  return k(src, tgt, idx)


def _solve_call(partials):
  def body(p_ref, o_ref):
    s = jnp.sum(p_ref[...], axis=0)
    inv = 1.0 / jnp.maximum(s[0], 1.0)
    ss = (s[1], s[2], s[3])
    st = (s[4], s[5], s[6])
    ms = tuple(a * inv for a in ss)
    mt = tuple(a * inv for a in st)
    H = [[s[7 + 3 * a + b] - ss[a] * st[b] * inv for b in range(3)]
         for a in range(3)]
    (Sxx, Sxy, Sxz), (Syx, Syy, Syz), (Szx, Szy, Szz) = H
    n00 = Sxx + Syy + Szz
    n01 = Syz - Szy
    n02 = Szx - Sxz
    n03 = Sxy - Syx
    n11 = Sxx - Syy - Szz
    n12 = Sxy + Syx
    n13 = Szx + Sxz
    n22 = -Sxx + Syy - Szz
    n23 = Syz + Szy
    n33 = -Sxx - Syy + Szz
    fro2 = sum(H[a][b] * H[a][b] for a in range(3) for b in range(3))
    shift = jnp.sqrt(3.0 * fro2) + 1e-30
    B = [[n00 + shift, n01, n02, n03],
         [n01, n11 + shift, n12, n13],
         [n02, n12, n22 + shift, n23],
         [n03, n13, n23, n33 + shift]]
    for _ in range(16):
      C = [[sum(B[i][k] * B[k][j] for k in range(4)) for j in range(4)]
           for i in range(4)]
      invtr = 1.0 / jnp.maximum(C[0][0] + C[1][1] + C[2][2] + C[3][3], 1e-30)
      B = [[C[i][j] * invtr for j in range(4)] for i in range(4)]
    d = [B[i][i] for i in range(4)]
    m0 = (d[0] >= d[1]) & (d[0] >= d[2]) & (d[0] >= d[3])
    m1 = (d[1] >= d[2]) & (d[1] >= d[3])
    m2 = d[2] >= d[3]
    q = [jnp.where(m0, B[i][0],
         jnp.where(m1, B[i][1],
         jnp.where(m2, B[i][2], B[i][3]))) for i in range(4)]
    qn = 1.0 / jnp.sqrt(q[0] * q[0] + q[1] * q[1] + q[2] * q[2]
                        + q[3] * q[3] + 1e-30)
    w, x, y, z = (qi * qn for qi in q)
    r = [1.0 - 2.0 * (y * y + z * z), 2.0 * (x * y - w * z), 2.0 * (x * z + w * y),
         2.0 * (x * y + w * z), 1.0 - 2.0 * (x * x + z * z), 2.0 * (y * z - w * x),
         2.0 * (x * z - w * y), 2.0 * (y * z + w * x), 1.0 - 2.0 * (x * x + y * y)]
    t = [mt[a] - (r[3 * a] * ms[0] + r[3 * a + 1] * ms[1] + r[3 * a + 2] * ms[2])
         for a in range(3)]
    for j in range(9):
      o_ref[j, :] = r[j]
    for a in range(3):
      o_ref[9 + a, :] = t[a]
    zero = jnp.zeros((NSEG,), jnp.float32)
    for j in range(12, 16):
      o_ref[j, :] = zero

  return pl.pallas_call(
      body,
      out_shape=jax.ShapeDtypeStruct((16, NSEG), jnp.float32),
  )(partials)


def _apply_call(src, idx, rt):
  n = idx.shape[0]
  ppt = n // NW
  nsub = ppt // CHUNK
  ngrp = CHUNK // LANES
  mesh = plsc.VectorSubcoreMesh(core_axis_name="c", subcore_axis_name="s")

  @functools.partial(
      pl.kernel, mesh=mesh,
      compiler_params=pltpu.CompilerParams(needs_layout_passes=False, use_tc_tiling_on_sc=False),
      out_type=jax.ShapeDtypeStruct((n, 3), jnp.float32),
      scratch_types=[
          pltpu.VMEM((CHUNK, 3), jnp.float32),
          pltpu.VMEM((CHUNK,), jnp.int32),
          pltpu.VMEM((CHUNK, 3), jnp.float32),
          pltpu.VMEM((16, NSEG), jnp.float32),
      ],
  )
  def k(src_h, idx_h, rt_h, out_h, sv, iv, ov, rtv):
    wid = lax.axis_index("s") * NC + lax.axis_index("c")
    iota = lax.iota(jnp.int32, LANES)
    c0 = jnp.zeros((LANES,), jnp.int32)
    c1 = jnp.full((LANES,), 1, jnp.int32)
    c2 = jnp.full((LANES,), 2, jnp.int32)
    jrows = [jnp.full((LANES,), j, jnp.int32) for j in range(12)]
    pltpu.sync_copy(rt_h, rtv)

    def sub_body(sub, carry):
      base = wid * ppt + sub * CHUNK
      pltpu.sync_copy(src_h.at[pl.ds(base, CHUNK)], sv)
      pltpu.sync_copy(idx_h.at[pl.ds(base, CHUNK)], iv)

      def grp_body(g, c):
        rows = g * LANES + iota
        ivec = iv[pl.ds(g * LANES, LANES)]
        sx = plsc.load_gather(sv, [rows, c0])
        sy = plsc.load_gather(sv, [rows, c1])
        sz = plsc.load_gather(sv, [rows, c2])
        coef = [plsc.load_gather(rtv, [jrows[j], ivec]) for j in range(12)]
        ax = coef[0] * sx + coef[1] * sy + coef[2] * sz + coef[9]
        ay = coef[3] * sx + coef[4] * sy + coef[5] * sz + coef[10]
        az = coef[6] * sx + coef[7] * sy + coef[8] * sz + coef[11]
        plsc.store_scatter(ov, [rows, c0], ax)
        plsc.store_scatter(ov, [rows, c1], ay)
        plsc.store_scatter(ov, [rows, c2], az)
        return c
      lax.fori_loop(0, ngrp, grp_body, 0)
      pltpu.sync_copy(ov, out_h.at[pl.ds(base, CHUNK)])
      return carry
    lax.fori_loop(0, nsub, sub_body, 0)

  return k(src, idx, rt)


def kernel(src_points, tgt_points, batch_indices):
  src = src_points.astype(jnp.float32)
  tgt = tgt_points.astype(jnp.float32)
  idx = batch_indices.astype(jnp.int32)
  partials = _pass1_call(src, tgt, idx)
  rt = _solve_call(partials)
  aligned = _apply_call(src, idx, rt)
  R = jnp.transpose(rt[:9]).reshape(NSEG, 3, 3)
  t = jnp.transpose(rt[9:12])
  return (aligned, (R, t))


# speedup vs baseline: 5.6782x; 5.6782x over previous
import functools

import jax
import jax.numpy as jnp
from jax import lax
from jax.experimental import pallas as pl
from jax.experimental.pallas import tpu as pltpu
from jax.experimental.pallas import tpu_sc as plsc

NSEG = 2048
NC = 2
NS = 16
NW = NC * NS
LANES = 16
CHUNK = 2048
CROWS = CHUNK // 128 * 4
TABLE = 16 * NSEG


def _to_rows(pts, n):
  p = jnp.pad(pts, ((0, 0), (0, 1)))
  return p.reshape(n // 128, 128, 4).swapaxes(1, 2).reshape(n // 32, 128)


def _pass1_call(srcx, tgtx, idx):
  n = idx.shape[0]
  ppt = n // NW
  nsub = ppt // CHUNK
  nblk = CHUNK // 128
  mesh = plsc.VectorSubcoreMesh(core_axis_name="c", subcore_axis_name="s")

  @functools.partial(
      pl.kernel, mesh=mesh,
      compiler_params=pltpu.CompilerParams(needs_layout_passes=False),
      out_type=jax.ShapeDtypeStruct((NW, TABLE), jnp.float32),
      scratch_types=[
          pltpu.VMEM((CROWS, 128), jnp.float32),
          pltpu.VMEM((CROWS, 128), jnp.float32),
          pltpu.VMEM((CHUNK,), jnp.int32),
          pltpu.VMEM((TABLE,), jnp.float32),
      ],
  )
  def k(src_h, tgt_h, idx_h, out_h, sv, tv, iv, tab):
    wid = lax.axis_index("s") * NC + lax.axis_index("c")
    ones = jnp.ones((LANES,), jnp.float32)
    zeros16 = jnp.zeros((LANES,), jnp.float32)

    def zero_body(i, carry):
      tab[pl.ds(i * LANES, LANES)] = zeros16
      return carry
    lax.fori_loop(0, TABLE // LANES, zero_body, 0)

    def sub_body(sub, carry):
      rbase = pl.multiple_of((wid * ppt + sub * CHUNK) // 128 * 4, 64)
      pltpu.sync_copy(src_h.at[pl.ds(rbase, CROWS)], sv)
      pltpu.sync_copy(tgt_h.at[pl.ds(rbase, CROWS)], tv)
      pltpu.sync_copy(idx_h.at[pl.ds(wid * ppt + sub * CHUNK, CHUNK)], iv)

      def blk_body(b, c):
        r = b * 4
        for gg in range(8):
          l = gg * LANES
          ivec = iv[pl.ds(b * 128 + l, LANES)]
          sx = sv[r, pl.ds(l, LANES)]
          sy = sv[r + 1, pl.ds(l, LANES)]
          sz = sv[r + 2, pl.ds(l, LANES)]
          tx = tv[r, pl.ds(l, LANES)]
          ty = tv[r + 1, pl.ds(l, LANES)]
          tz = tv[r + 2, pl.ds(l, LANES)]
          vals = (ones, sx, sy, sz, tx, ty, tz,
                  sx * tx, sx * ty, sx * tz,
                  sy * tx, sy * ty, sy * tz,
                  sz * tx, sz * ty, sz * tz)
          for j, v in enumerate(vals):
            plsc.addupdate_scatter(tab, [ivec + j * NSEG], v)
        return c
      lax.fori_loop(0, nblk, blk_body, 0)
      return carry
    lax.fori_loop(0, nsub, sub_body, 0)
    pltpu.sync_copy(tab, out_h.at[wid])

  return k(srcx, tgtx, idx)


def _solve_call(partials):
  def body(p_ref, o_ref):
    s = jnp.sum(p_ref[...], axis=0)
    inv = 1.0 / jnp.maximum(s[0], 1.0)
    ss = (s[1], s[2], s[3])
    st = (s[4], s[5], s[6])
    ms = tuple(a * inv for a in ss)
    mt = tuple(a * inv for a in st)
    H = [[s[7 + 3 * a + b] - ss[a] * st[b] * inv for b in range(3)]
         for a in range(3)]
    (Sxx, Sxy, Sxz), (Syx, Syy, Syz), (Szx, Szy, Szz) = H
    n00 = Sxx + Syy + Szz
    n01 = Syz - Szy
    n02 = Szx - Sxz
    n03 = Sxy - Syx
    n11 = Sxx - Syy - Szz
    n12 = Sxy + Syx
    n13 = Szx + Sxz
    n22 = -Sxx + Syy - Szz
    n23 = Syz + Szy
    n33 = -Sxx - Syy + Szz
    fro2 = sum(H[a][b] * H[a][b] for a in range(3) for b in range(3))
    shift = jnp.sqrt(3.0 * fro2) + 1e-30
    B = [[n00 + shift, n01, n02, n03],
         [n01, n11 + shift, n12, n13],
         [n02, n12, n22 + shift, n23],
         [n03, n13, n23, n33 + shift]]
    for _ in range(16):
      C = [[sum(B[i][k] * B[k][j] for k in range(4)) for j in range(4)]
           for i in range(4)]
      invtr = 1.0 / jnp.maximum(C[0][0] + C[1][1] + C[2][2] + C[3][3], 1e-30)
      B = [[C[i][j] * invtr for j in range(4)] for i in range(4)]
    d = [B[i][i] for i in range(4)]
    m0 = (d[0] >= d[1]) & (d[0] >= d[2]) & (d[0] >= d[3])
    m1 = (d[1] >= d[2]) & (d[1] >= d[3])
    m2 = d[2] >= d[3]
    q = [jnp.where(m0, B[i][0],
         jnp.where(m1, B[i][1],
         jnp.where(m2, B[i][2], B[i][3]))) for i in range(4)]
    qn = 1.0 / jnp.sqrt(q[0] * q[0] + q[1] * q[1] + q[2] * q[2]
                        + q[3] * q[3] + 1e-30)
    w, x, y, z = (qi * qn for qi in q)
    r = [1.0 - 2.0 * (y * y + z * z), 2.0 * (x * y - w * z), 2.0 * (x * z + w * y),
         2.0 * (x * y + w * z), 1.0 - 2.0 * (x * x + z * z), 2.0 * (y * z - w * x),
         2.0 * (x * z - w * y), 2.0 * (y * z + w * x), 1.0 - 2.0 * (x * x + y * y)]
    t = [mt[a] - (r[3 * a] * ms[0] + r[3 * a + 1] * ms[1] + r[3 * a + 2] * ms[2])
         for a in range(3)]
    for j in range(9):
      o_ref[j, :] = r[j]
    for a in range(3):
      o_ref[9 + a, :] = t[a]
    zero = jnp.zeros((NSEG,), jnp.float32)
    for j in range(12, 16):
      o_ref[j, :] = zero

  return pl.pallas_call(
      body,
      out_shape=jax.ShapeDtypeStruct((16, NSEG), jnp.float32),
  )(partials)


def _apply_call(srcx, idx, rt_flat):
  n = idx.shape[0]
  ppt = n // NW
  nsub = ppt // CHUNK
  nblk = CHUNK // 128
  mesh = plsc.VectorSubcoreMesh(core_axis_name="c", subcore_axis_name="s")

  @functools.partial(
      pl.kernel, mesh=mesh,
      compiler_params=pltpu.CompilerParams(needs_layout_passes=False),
      out_type=jax.ShapeDtypeStruct((n // 32, 128), jnp.float32),
      scratch_types=[
          pltpu.VMEM((CROWS, 128), jnp.float32),
          pltpu.VMEM((CHUNK,), jnp.int32),
          pltpu.VMEM((CROWS, 128), jnp.float32),
          pltpu.VMEM((12 * NSEG,), jnp.float32),
      ],
  )
  def k(src_h, idx_h, rt_h, out_h, sv, iv, ov, rtv):
    wid = lax.axis_index("s") * NC + lax.axis_index("c")
    pltpu.sync_copy(rt_h, rtv)

    def sub_body(sub, carry):
      rbase = pl.multiple_of((wid * ppt + sub * CHUNK) // 128 * 4, 64)
      pltpu.sync_copy(src_h.at[pl.ds(rbase, CROWS)], sv)
      pltpu.sync_copy(idx_h.at[pl.ds(wid * ppt + sub * CHUNK, CHUNK)], iv)

      def blk_body(b, c):
        r = b * 4
        for gg in range(8):
          l = gg * LANES
          ivec = iv[pl.ds(b * 128 + l, LANES)]
          sx = sv[r, pl.ds(l, LANES)]
          sy = sv[r + 1, pl.ds(l, LANES)]
          sz = sv[r + 2, pl.ds(l, LANES)]
          coef = [plsc.load_gather(rtv, [ivec + j * NSEG]) for j in range(12)]
          ov[r, pl.ds(l, LANES)] = (
              coef[0] * sx + coef[1] * sy + coef[2] * sz + coef[9])
          ov[r + 1, pl.ds(l, LANES)] = (
              coef[3] * sx + coef[4] * sy + coef[5] * sz + coef[10])
          ov[r + 2, pl.ds(l, LANES)] = (
              coef[6] * sx + coef[7] * sy + coef[8] * sz + coef[11])
        return c
      lax.fori_loop(0, nblk, blk_body, 0)
      pltpu.sync_copy(ov, out_h.at[pl.ds(rbase, CROWS)])
      return carry
    lax.fori_loop(0, nsub, sub_body, 0)

  return k(srcx, idx, rt_flat)


def kernel(src_points, tgt_points, batch_indices):
  n = src_points.shape[0]
  src = src_points.astype(jnp.float32)
  tgt = tgt_points.astype(jnp.float32)
  idx = batch_indices.astype(jnp.int32)
  srcx = _to_rows(src, n)
  tgtx = _to_rows(tgt, n)
  partials = _pass1_call(srcx, tgtx, idx)
  rt = _solve_call(partials.reshape(NW, 16, NSEG))
  alignedx = _apply_call(srcx, idx, rt[:12].reshape(12 * NSEG))
  aligned = alignedx.reshape(n // 128, 4, 128).swapaxes(1, 2).reshape(n, 4)[:, :3]
  R = jnp.transpose(rt[:9]).reshape(NSEG, 3, 3)
  t = jnp.transpose(rt[9:12])
  return (aligned, (R, t))


# speedup vs baseline: 16.7476x; 2.9495x over previous
import functools

import jax
import jax.numpy as jnp
from jax import lax
from jax.experimental import pallas as pl
from jax.experimental.pallas import tpu as pltpu
from jax.experimental.pallas import tpu_sc as plsc

NSEG = 2048
NC = 2
NS = 16
NW = NC * NS
LANES = 16
CHUNK = 2048
CROWS = CHUNK // 128 * 4
TABLE = 16 * NSEG


def _to_rows(pts, n):
  p = jnp.pad(pts, ((0, 0), (0, 1)))
  return p.reshape(n // 128, 128, 4).swapaxes(1, 2).reshape(n // 32, 128)


def _pass1_call(srcx, tgtx, idx):
  n = idx.shape[0]
  ppt = n // NW
  nsub = ppt // CHUNK
  nblk = CHUNK // 128
  mesh = plsc.VectorSubcoreMesh(core_axis_name="c", subcore_axis_name="s")

  @functools.partial(
      pl.kernel, mesh=mesh,
      compiler_params=pltpu.CompilerParams(needs_layout_passes=False),
      out_type=jax.ShapeDtypeStruct((NW, TABLE), jnp.float32),
      scratch_types=[
          pltpu.VMEM((CROWS, 128), jnp.float32),
          pltpu.VMEM((CROWS, 128), jnp.float32),
          pltpu.VMEM((CHUNK,), jnp.int32),
          pltpu.VMEM((TABLE,), jnp.float32),
      ],
  )
  def k(src_h, tgt_h, idx_h, out_h, sv, tv, iv, tab):
    wid = lax.axis_index("s") * NC + lax.axis_index("c")
    ones = jnp.ones((LANES,), jnp.float32)
    zeros16 = jnp.zeros((LANES,), jnp.float32)
    iota = lax.iota(jnp.int32, LANES)
    iota_seg = iota * NSEG

    def zero_body(i, carry):
      tab[pl.ds(i * LANES, LANES)] = zeros16
      return carry
    lax.fori_loop(0, TABLE // LANES, zero_body, 0)

    def flush(cur, accs):
      @pl.when(cur >= 0)
      def _():
        row = zeros16
        for j in range(16):
          row = jnp.where(iota == j, jnp.sum(accs[j]), row)
        plsc.addupdate_scatter(tab, [iota_seg + cur], row)

    def sub_body(sub, carry):
      rbase = pl.multiple_of((wid * ppt + sub * CHUNK) // 128 * 4, 64)
      pltpu.sync_copy(src_h.at[pl.ds(rbase, CROWS)], sv)
      pltpu.sync_copy(tgt_h.at[pl.ds(rbase, CROWS)], tv)
      pltpu.sync_copy(idx_h.at[pl.ds(wid * ppt + sub * CHUNK, CHUNK)], iv)

      def grp_body(g, c):
        cur = c[0]
        accs = c[1:]
        r = g // 8 * 4
        l = g % 8 * LANES
        ivec = iv[pl.ds(g * LANES, LANES)]
        sx = sv[r, pl.ds(l, LANES)]
        sy = sv[r + 1, pl.ds(l, LANES)]
        sz = sv[r + 2, pl.ds(l, LANES)]
        tx = tv[r, pl.ds(l, LANES)]
        ty = tv[r + 1, pl.ds(l, LANES)]
        tz = tv[r + 2, pl.ds(l, LANES)]
        vals = (ones, sx, sy, sz, tx, ty, tz,
                sx * tx, sx * ty, sx * tz,
                sy * tx, sy * ty, sy * tz,
                sz * tx, sz * ty, sz * tz)
        smin = jnp.min(ivec)
        smax = jnp.max(ivec)

        def fast_path():
          return (cur,) + tuple(a + v for a, v in zip(accs, vals))

        def new_run():
          flush(cur, accs)
          return (smin,) + vals

        def mixed():
          flush(cur, accs)
          for j, v in enumerate(vals):
            plsc.addupdate_scatter(tab, [ivec + j * NSEG], v)
          return (smax,) + (zeros16,) * 16

        def slow_path():
          return lax.cond(smin == smax, new_run, mixed)

        return lax.cond((smin == cur) & (smax == cur), fast_path, slow_path)

      return lax.fori_loop(0, CHUNK // LANES, grp_body, carry)

    init = (jnp.int32(-1),) + (zeros16,) * 16
    fin = lax.fori_loop(0, nsub, sub_body, init)
    flush(fin[0], fin[1:])
    pltpu.sync_copy(tab, out_h.at[wid])

  return k(srcx, tgtx, idx)


def _solve_call(partials):
  def body(p_ref, o_ref):
    s = jnp.sum(p_ref[...], axis=0)
    inv = 1.0 / jnp.maximum(s[0], 1.0)
    ss = (s[1], s[2], s[3])
    st = (s[4], s[5], s[6])
    ms = tuple(a * inv for a in ss)
    mt = tuple(a * inv for a in st)
    H = [[s[7 + 3 * a + b] - ss[a] * st[b] * inv for b in range(3)]
         for a in range(3)]
    (Sxx, Sxy, Sxz), (Syx, Syy, Syz), (Szx, Szy, Szz) = H
    n00 = Sxx + Syy + Szz
    n01 = Syz - Szy
    n02 = Szx - Sxz
    n03 = Sxy - Syx
    n11 = Sxx - Syy - Szz
    n12 = Sxy + Syx
    n13 = Szx + Sxz
    n22 = -Sxx + Syy - Szz
    n23 = Syz + Szy
    n33 = -Sxx - Syy + Szz
    fro2 = sum(H[a][b] * H[a][b] for a in range(3) for b in range(3))
    shift = jnp.sqrt(3.0 * fro2) + 1e-30
    B = [[n00 + shift, n01, n02, n03],
         [n01, n11 + shift, n12, n13],
         [n02, n12, n22 + shift, n23],
         [n03, n13, n23, n33 + shift]]
    for _ in range(16):
      C = [[sum(B[i][k] * B[k][j] for k in range(4)) for j in range(4)]
           for i in range(4)]
      invtr = 1.0 / jnp.maximum(C[0][0] + C[1][1] + C[2][2] + C[3][3], 1e-30)
      B = [[C[i][j] * invtr for j in range(4)] for i in range(4)]
    d = [B[i][i] for i in range(4)]
    m0 = (d[0] >= d[1]) & (d[0] >= d[2]) & (d[0] >= d[3])
    m1 = (d[1] >= d[2]) & (d[1] >= d[3])
    m2 = d[2] >= d[3]
    q = [jnp.where(m0, B[i][0],
         jnp.where(m1, B[i][1],
         jnp.where(m2, B[i][2], B[i][3]))) for i in range(4)]
    qn = 1.0 / jnp.sqrt(q[0] * q[0] + q[1] * q[1] + q[2] * q[2]
                        + q[3] * q[3] + 1e-30)
    w, x, y, z = (qi * qn for qi in q)
    r = [1.0 - 2.0 * (y * y + z * z), 2.0 * (x * y - w * z), 2.0 * (x * z + w * y),
         2.0 * (x * y + w * z), 1.0 - 2.0 * (x * x + z * z), 2.0 * (y * z - w * x),
         2.0 * (x * z - w * y), 2.0 * (y * z + w * x), 1.0 - 2.0 * (x * x + y * y)]
    t = [mt[a] - (r[3 * a] * ms[0] + r[3 * a + 1] * ms[1] + r[3 * a + 2] * ms[2])
         for a in range(3)]
    for j in range(9):
      o_ref[j, :] = r[j]
    for a in range(3):
      o_ref[9 + a, :] = t[a]
    zero = jnp.zeros((NSEG,), jnp.float32)
    for j in range(12, 16):
      o_ref[j, :] = zero

  return pl.pallas_call(
      body,
      out_shape=jax.ShapeDtypeStruct((16, NSEG), jnp.float32),
  )(partials)


def _apply_call(srcx, idx, rt_flat):
  n = idx.shape[0]
  ppt = n // NW
  nsub = ppt // CHUNK
  nblk = CHUNK // 128
  mesh = plsc.VectorSubcoreMesh(core_axis_name="c", subcore_axis_name="s")

  @functools.partial(
      pl.kernel, mesh=mesh,
      compiler_params=pltpu.CompilerParams(needs_layout_passes=False),
      out_type=jax.ShapeDtypeStruct((n // 32, 128), jnp.float32),
      scratch_types=[
          pltpu.VMEM((CROWS, 128), jnp.float32),
          pltpu.VMEM((CHUNK,), jnp.int32),
          pltpu.VMEM((CROWS, 128), jnp.float32),
          pltpu.VMEM((12 * NSEG,), jnp.float32),
      ],
  )
  def k(src_h, idx_h, rt_h, out_h, sv, iv, ov, rtv):
    wid = lax.axis_index("s") * NC + lax.axis_index("c")
    pltpu.sync_copy(rt_h, rtv)

    def sub_body(sub, carry):
      rbase = pl.multiple_of((wid * ppt + sub * CHUNK) // 128 * 4, 64)
      pltpu.sync_copy(src_h.at[pl.ds(rbase, CROWS)], sv)
      pltpu.sync_copy(idx_h.at[pl.ds(wid * ppt + sub * CHUNK, CHUNK)], iv)

      def blk_body(b, c):
        r = b * 4
        for gg in range(8):
          l = gg * LANES
          ivec = iv[pl.ds(b * 128 + l, LANES)]
          sx = sv[r, pl.ds(l, LANES)]
          sy = sv[r + 1, pl.ds(l, LANES)]
          sz = sv[r + 2, pl.ds(l, LANES)]
          coef = [plsc.load_gather(rtv, [ivec + j * NSEG]) for j in range(12)]
          ov[r, pl.ds(l, LANES)] = (
              coef[0] * sx + coef[1] * sy + coef[2] * sz + coef[9])
          ov[r + 1, pl.ds(l, LANES)] = (
              coef[3] * sx + coef[4] * sy + coef[5] * sz + coef[10])
          ov[r + 2, pl.ds(l, LANES)] = (
              coef[6] * sx + coef[7] * sy + coef[8] * sz + coef[11])
        return c
      lax.fori_loop(0, nblk, blk_body, 0)
      pltpu.sync_copy(ov, out_h.at[pl.ds(rbase, CROWS)])
      return carry
    lax.fori_loop(0, nsub, sub_body, 0)

  return k(srcx, idx, rt_flat)


def kernel(src_points, tgt_points, batch_indices):
  n = src_points.shape[0]
  src = src_points.astype(jnp.float32)
  tgt = tgt_points.astype(jnp.float32)
  idx = batch_indices.astype(jnp.int32)
  srcx = _to_rows(src, n)
  tgtx = _to_rows(tgt, n)
  partials = _pass1_call(srcx, tgtx, idx)
  rt = _solve_call(partials.reshape(NW, 16, NSEG))
  alignedx = _apply_call(srcx, idx, rt[:12].reshape(12 * NSEG))
  aligned = alignedx.reshape(n // 128, 4, 128).swapaxes(1, 2).reshape(n, 4)[:, :3]
  R = jnp.transpose(rt[:9]).reshape(NSEG, 3, 3)
  t = jnp.transpose(rt[9:12])
  return (aligned, (R, t))


# speedup vs baseline: 21.9131x; 1.3084x over previous
import functools

import jax
import jax.numpy as jnp
from jax import lax
from jax.experimental import pallas as pl
from jax.experimental.pallas import tpu as pltpu
from jax.experimental.pallas import tpu_sc as plsc

NSEG = 2048
NC = 2
NS = 16
NW = NC * NS
LANES = 16
CHUNK = 1024
CROWS = CHUNK // 128 * 4
TABLE = 16 * NSEG


def _to_rows(pts, n):
  p = jnp.pad(pts, ((0, 0), (0, 1)))
  return p.reshape(n // 128, 128, 4).swapaxes(1, 2).reshape(n // 32, 128)


def _pass1_call(srcx, tgtx, idx):
  n = idx.shape[0]
  ppt = n // NW
  nsub = ppt // CHUNK
  mesh = plsc.VectorSubcoreMesh(core_axis_name="c", subcore_axis_name="s")

  @functools.partial(
      pl.kernel, mesh=mesh,
      compiler_params=pltpu.CompilerParams(needs_layout_passes=False),
      out_type=jax.ShapeDtypeStruct((NW, TABLE), jnp.float32),
      scratch_types=[
          pltpu.VMEM((CROWS, 128), jnp.float32),
          pltpu.VMEM((CROWS, 128), jnp.float32),
          pltpu.VMEM((CROWS, 128), jnp.float32),
          pltpu.VMEM((CROWS, 128), jnp.float32),
          pltpu.VMEM((CHUNK,), jnp.int32),
          pltpu.VMEM((CHUNK,), jnp.int32),
          pltpu.VMEM((TABLE,), jnp.float32),
          pltpu.SemaphoreType.DMA,
          pltpu.SemaphoreType.DMA,
      ],
  )
  def k(src_h, tgt_h, idx_h, out_h, sv0, tv0, sv1, tv1, iv0, iv1,
        tab, semA, semB):
    wid = lax.axis_index("s") * NC + lax.axis_index("c")
    ones = jnp.ones((LANES,), jnp.float32)
    zeros16 = jnp.zeros((LANES,), jnp.float32)
    iota = lax.iota(jnp.int32, LANES)
    iota_seg = iota * NSEG

    def zero_body(i, carry):
      for j in range(8):
        tab[pl.ds(i * (8 * LANES) + j * LANES, LANES)] = zeros16
      return carry
    lax.fori_loop(0, TABLE // (8 * LANES), zero_body, 0)

    def copies(sub, sv_, tv_, iv_, sem):
      rbase = pl.multiple_of((wid * ppt + sub * CHUNK) // 128 * 4, CROWS)
      ibase = wid * ppt + sub * CHUNK
      return (
          pltpu.make_async_copy(src_h.at[pl.ds(rbase, CROWS)], sv_, sem),
          pltpu.make_async_copy(tgt_h.at[pl.ds(rbase, CROWS)], tv_, sem),
          pltpu.make_async_copy(idx_h.at[pl.ds(ibase, CHUNK)], iv_, sem),
      )

    def issue(sub, sv_, tv_, iv_, sem):
      for cpy in copies(sub, sv_, tv_, iv_, sem):
        cpy.start()

    def drain(sub, sv_, tv_, iv_, sem):
      for cpy in copies(sub, sv_, tv_, iv_, sem):
        cpy.wait()

    def flush(cur, accs):
      @pl.when(cur >= 0)
      def _():
        row = zeros16
        for j in range(16):
          row = jnp.where(iota == j, jnp.sum(accs[j]), row)
        plsc.addupdate_scatter(tab, [iota_seg + cur], row)

    def process(sv_, tv_, iv_, carry):

      def grp_body(g, c):
        cur = c[0]
        accs = c[1:]
        r = g // 8 * 4
        l = g % 8 * LANES
        ivec = iv_[pl.ds(g * LANES, LANES)]
        first = ivec[0]
        last = ivec[LANES - 1]
        sx = sv_[r, pl.ds(l, LANES)]
        sy = sv_[r + 1, pl.ds(l, LANES)]
        sz = sv_[r + 2, pl.ds(l, LANES)]
        tx = tv_[r, pl.ds(l, LANES)]
        ty = tv_[r + 1, pl.ds(l, LANES)]
        tz = tv_[r + 2, pl.ds(l, LANES)]
        vals = (ones, sx, sy, sz, tx, ty, tz,
                sx * tx, sx * ty, sx * tz,
                sy * tx, sy * ty, sy * tz,
                sz * tx, sz * ty, sz * tz)

        def fast_path():
          return (cur,) + tuple(a + v for a, v in zip(accs, vals))

        def new_run():
          flush(cur, accs)
          return (first,) + vals

        def mixed():
          flush(cur, accs)
          for j, v in enumerate(vals):
            plsc.addupdate_scatter(tab, [ivec + j * NSEG], v)
          return (last,) + (zeros16,) * 16

        def slow_path():
          return lax.cond(first == last, new_run, mixed)

        return lax.cond((first == cur) & (last == cur), fast_path, slow_path)

      return lax.fori_loop(0, CHUNK // LANES, grp_body, carry)

    issue(0, sv0, tv0, iv0, semA)

    def pair_body(it, carry):
      sub0 = it * 2
      issue(sub0 + 1, sv1, tv1, iv1, semB)
      drain(sub0, sv0, tv0, iv0, semA)
      carry = process(sv0, tv0, iv0, carry)

      @pl.when(sub0 + 2 < nsub)
      def _():
        issue(sub0 + 2, sv0, tv0, iv0, semA)
      drain(sub0 + 1, sv1, tv1, iv1, semB)
      return process(sv1, tv1, iv1, carry)

    init = (jnp.int32(-1),) + (zeros16,) * 16
    fin = lax.fori_loop(0, nsub // 2, pair_body, init)
    flush(fin[0], fin[1:])
    pltpu.sync_copy(tab, out_h.at[wid])

  return k(srcx, tgtx, idx)


def _solve_call(partials):
  def body(p_ref, o_ref):
    s = jnp.sum(p_ref[...], axis=0)
    inv = 1.0 / jnp.maximum(s[0], 1.0)
    ss = (s[1], s[2], s[3])
    st = (s[4], s[5], s[6])
    ms = tuple(a * inv for a in ss)
    mt = tuple(a * inv for a in st)
    H = [[s[7 + 3 * a + b] - ss[a] * st[b] * inv for b in range(3)]
         for a in range(3)]
    (Sxx, Sxy, Sxz), (Syx, Syy, Syz), (Szx, Szy, Szz) = H
    n00 = Sxx + Syy + Szz
    n01 = Syz - Szy
    n02 = Szx - Sxz
    n03 = Sxy - Syx
    n11 = Sxx - Syy - Szz
    n12 = Sxy + Syx
    n13 = Szx + Sxz
    n22 = -Sxx + Syy - Szz
    n23 = Syz + Szy
    n33 = -Sxx - Syy + Szz
    fro2 = sum(H[a][b] * H[a][b] for a in range(3) for b in range(3))
    shift = jnp.sqrt(3.0 * fro2) + 1e-30
    B = [[n00 + shift, n01, n02, n03],
         [n01, n11 + shift, n12, n13],
         [n02, n12, n22 + shift, n23],
         [n03, n13, n23, n33 + shift]]
    for _ in range(16):
      C = [[sum(B[i][k] * B[k][j] for k in range(4)) for j in range(4)]
           for i in range(4)]
      invtr = 1.0 / jnp.maximum(C[0][0] + C[1][1] + C[2][2] + C[3][3], 1e-30)
      B = [[C[i][j] * invtr for j in range(4)] for i in range(4)]
    d = [B[i][i] for i in range(4)]
    m0 = (d[0] >= d[1]) & (d[0] >= d[2]) & (d[0] >= d[3])
    m1 = (d[1] >= d[2]) & (d[1] >= d[3])
    m2 = d[2] >= d[3]
    q = [jnp.where(m0, B[i][0],
         jnp.where(m1, B[i][1],
         jnp.where(m2, B[i][2], B[i][3]))) for i in range(4)]
    qn = 1.0 / jnp.sqrt(q[0] * q[0] + q[1] * q[1] + q[2] * q[2]
                        + q[3] * q[3] + 1e-30)
    w, x, y, z = (qi * qn for qi in q)
    r = [1.0 - 2.0 * (y * y + z * z), 2.0 * (x * y - w * z), 2.0 * (x * z + w * y),
         2.0 * (x * y + w * z), 1.0 - 2.0 * (x * x + z * z), 2.0 * (y * z - w * x),
         2.0 * (x * z - w * y), 2.0 * (y * z + w * x), 1.0 - 2.0 * (x * x + y * y)]
    t = [mt[a] - (r[3 * a] * ms[0] + r[3 * a + 1] * ms[1] + r[3 * a + 2] * ms[2])
         for a in range(3)]
    for j in range(9):
      o_ref[j, :] = r[j]
    for a in range(3):
      o_ref[9 + a, :] = t[a]
    zero = jnp.zeros((NSEG,), jnp.float32)
    for j in range(12, 16):
      o_ref[j, :] = zero

  return pl.pallas_call(
      body,
      out_shape=jax.ShapeDtypeStruct((16, NSEG), jnp.float32),
  )(partials)


def _apply_call(srcx, idx, rt_flat):
  n = idx.shape[0]
  ppt = n // NW
  nsub = ppt // CHUNK
  mesh = plsc.VectorSubcoreMesh(core_axis_name="c", subcore_axis_name="s")

  @functools.partial(
      pl.kernel, mesh=mesh,
      compiler_params=pltpu.CompilerParams(needs_layout_passes=False),
      out_type=jax.ShapeDtypeStruct((n // 32, 128), jnp.float32),
      scratch_types=[
          pltpu.VMEM((CROWS, 128), jnp.float32),
          pltpu.VMEM((CROWS, 128), jnp.float32),
          pltpu.VMEM((CHUNK,), jnp.int32),
          pltpu.VMEM((CHUNK,), jnp.int32),
          pltpu.VMEM((CROWS, 128), jnp.float32),
          pltpu.VMEM((CROWS, 128), jnp.float32),
          pltpu.VMEM((12 * NSEG,), jnp.float32),
          pltpu.SemaphoreType.DMA,
          pltpu.SemaphoreType.DMA,
          pltpu.SemaphoreType.DMA,
          pltpu.SemaphoreType.DMA,
      ],
  )
  def k(src_h, idx_h, rt_h, out_h, sv0, sv1, iv0, iv1, ov0, ov1,
        rtv, semA, semB, semC, semD):
    wid = lax.axis_index("s") * NC + lax.axis_index("c")
    pltpu.sync_copy(rt_h, rtv)
    zeros16 = jnp.zeros((LANES,), jnp.float32)

    def rb(sub):
      return pl.multiple_of((wid * ppt + sub * CHUNK) // 128 * 4, CROWS)

    def copies(sub, sv_, iv_, sem):
      ibase = wid * ppt + sub * CHUNK
      return (
          pltpu.make_async_copy(src_h.at[pl.ds(rb(sub), CROWS)], sv_, sem),
          pltpu.make_async_copy(idx_h.at[pl.ds(ibase, CHUNK)], iv_, sem),
      )

    def issue(sub, sv_, iv_, sem):
      for cpy in copies(sub, sv_, iv_, sem):
        cpy.start()

    def drain(sub, sv_, iv_, sem):
      for cpy in copies(sub, sv_, iv_, sem):
        cpy.wait()

    def out_copy(sub, ov_, sem):
      return pltpu.make_async_copy(ov_, out_h.at[pl.ds(rb(sub), CROWS)], sem)

    def process(sub, sv_, iv_, ov_, carry):

      def grp_body(g, c):
        cur = c[0]
        cf = c[1:]
        r = g // 8 * 4
        l = g % 8 * LANES
        ivec = iv_[pl.ds(g * LANES, LANES)]
        first = ivec[0]
        last = ivec[LANES - 1]
        sx = sv_[r, pl.ds(l, LANES)]
        sy = sv_[r + 1, pl.ds(l, LANES)]
        sz = sv_[r + 2, pl.ds(l, LANES)]

        def emit(co):
          ov_[r, pl.ds(l, LANES)] = co[0] * sx + co[1] * sy + co[2] * sz + co[9]
          ov_[r + 1, pl.ds(l, LANES)] = (
              co[3] * sx + co[4] * sy + co[5] * sz + co[10])
          ov_[r + 2, pl.ds(l, LANES)] = (
              co[6] * sx + co[7] * sy + co[8] * sz + co[11])

        def fast_path():
          emit(cf)
          return c

        def new_run():
          co = tuple(
              plsc.load_gather(rtv, [ivec + j * NSEG]) for j in range(12))
          emit(co)
          return (first,) + co

        def mixed():
          co = tuple(
              plsc.load_gather(rtv, [ivec + j * NSEG]) for j in range(12))
          emit(co)
          return (jnp.int32(-1),) + cf

        def slow_path():
          return lax.cond(first == last, new_run, mixed)

        return lax.cond((first == cur) & (last == cur), fast_path, slow_path)

      carry = lax.fori_loop(0, CHUNK // LANES, grp_body, carry)
      return carry

    issue(0, sv0, iv0, semA)

    def pair_body(it, carry):
      sub0 = it * 2
      issue(sub0 + 1, sv1, iv1, semB)
      drain(sub0, sv0, iv0, semA)

      @pl.when(it > 0)
      def _():
        out_copy(sub0 - 2, ov0, semC).wait()
      carry = process(sub0, sv0, iv0, ov0, carry)
      out_copy(sub0, ov0, semC).start()

      @pl.when(sub0 + 2 < nsub)
      def _():
        issue(sub0 + 2, sv0, iv0, semA)
      drain(sub0 + 1, sv1, iv1, semB)

      @pl.when(it > 0)
      def _():
        out_copy(sub0 - 1, ov1, semD).wait()
      carry = process(sub0 + 1, sv1, iv1, ov1, carry)
      out_copy(sub0 + 1, ov1, semD).start()
      return carry

    init = (jnp.int32(-1),) + (zeros16,) * 12
    lax.fori_loop(0, nsub // 2, pair_body, init)
    out_copy(nsub - 2, ov0, semC).wait()
    out_copy(nsub - 1, ov1, semD).wait()

  return k(srcx, idx, rt_flat)


def kernel(src_points, tgt_points, batch_indices):
  n = src_points.shape[0]
  src = src_points.astype(jnp.float32)
  tgt = tgt_points.astype(jnp.float32)
  idx = batch_indices.astype(jnp.int32)
  srcx = _to_rows(src, n)
  tgtx = _to_rows(tgt, n)
  partials = _pass1_call(srcx, tgtx, idx)
  rt = _solve_call(partials.reshape(NW, 16, NSEG))
  alignedx = _apply_call(srcx, idx, rt[:12].reshape(12 * NSEG))
  aligned = alignedx.reshape(n // 128, 4, 128).swapaxes(1, 2).reshape(n, 4)[:, :3]
  R = jnp.transpose(rt[:9]).reshape(NSEG, 3, 3)
  t = jnp.transpose(rt[9:12])
  return (aligned, (R, t))


# speedup vs baseline: 25.7499x; 1.1751x over previous
import functools

import jax
import jax.numpy as jnp
from jax import lax
from jax.experimental import pallas as pl
from jax.experimental.pallas import tpu as pltpu
from jax.experimental.pallas import tpu_sc as plsc

NSEG = 2048
NC = 2
NS = 16
NW = NC * NS
LANES = 16
CHUNK = 1024
CROWS = CHUNK // 128 * 4
TABLE = 16 * NSEG


def _to_rows(pts, n):
  p = jnp.pad(pts, ((0, 0), (0, 1)))
  return p.reshape(n // 128, 128, 4).swapaxes(1, 2).reshape(n // 32, 128)


def _pass1_call(srcx, tgtx, idx):
  n = idx.shape[0]
  ppt = n // NW
  nsub = ppt // CHUNK
  mesh = plsc.VectorSubcoreMesh(core_axis_name="c", subcore_axis_name="s")

  @functools.partial(
      pl.kernel, mesh=mesh,
      compiler_params=pltpu.CompilerParams(needs_layout_passes=False),
      out_type=jax.ShapeDtypeStruct((NW, TABLE), jnp.float32),
      scratch_types=[
          pltpu.VMEM((CROWS, 128), jnp.float32),
          pltpu.VMEM((CROWS, 128), jnp.float32),
          pltpu.VMEM((CROWS, 128), jnp.float32),
          pltpu.VMEM((CROWS, 128), jnp.float32),
          pltpu.VMEM((CHUNK,), jnp.int32),
          pltpu.VMEM((CHUNK,), jnp.int32),
          pltpu.VMEM((TABLE,), jnp.float32),
          pltpu.SemaphoreType.DMA,
          pltpu.SemaphoreType.DMA,
      ],
  )
  def k(src_h, tgt_h, idx_h, out_h, sv0, tv0, sv1, tv1, iv0, iv1,
        tab, semA, semB):
    wid = lax.axis_index("s") * NC + lax.axis_index("c")
    ones = jnp.ones((LANES,), jnp.float32)
    zeros16 = jnp.zeros((LANES,), jnp.float32)
    iota = lax.iota(jnp.int32, LANES)
    iota_seg = iota * NSEG

    def zero_body(i, carry):
      for j in range(8):
        tab[pl.ds(i * (8 * LANES) + j * LANES, LANES)] = zeros16
      return carry
    lax.fori_loop(0, TABLE // (8 * LANES), zero_body, 0)

    def copies(sub, sv_, tv_, iv_, sem):
      rbase = pl.multiple_of((wid * ppt + sub * CHUNK) // 128 * 4, CROWS)
      ibase = wid * ppt + sub * CHUNK
      return (
          pltpu.make_async_copy(src_h.at[pl.ds(rbase, CROWS)], sv_, sem),
          pltpu.make_async_copy(tgt_h.at[pl.ds(rbase, CROWS)], tv_, sem),
          pltpu.make_async_copy(idx_h.at[pl.ds(ibase, CHUNK)], iv_, sem),
      )

    def issue(sub, sv_, tv_, iv_, sem):
      for cpy in copies(sub, sv_, tv_, iv_, sem):
        cpy.start()

    def drain(sub, sv_, tv_, iv_, sem):
      for cpy in copies(sub, sv_, tv_, iv_, sem):
        cpy.wait()

    def flush(cur, accs):
      @pl.when(cur >= 0)
      def _():
        row = zeros16
        for j in range(16):
          row = jnp.where(iota == j, jnp.sum(accs[j]), row)
        plsc.addupdate_scatter(tab, [iota_seg + cur], row)

    def process(sv_, tv_, iv_, carry):

      def grp_body(g, c):
        cur = c[0]
        accs = c[1:]
        r = g // 8 * 4
        l = g % 8 * LANES
        ivec = iv_[pl.ds(g * LANES, LANES)]
        first = ivec[0]
        last = ivec[LANES - 1]
        sx = sv_[r, pl.ds(l, LANES)]
        sy = sv_[r + 1, pl.ds(l, LANES)]
        sz = sv_[r + 2, pl.ds(l, LANES)]
        tx = tv_[r, pl.ds(l, LANES)]
        ty = tv_[r + 1, pl.ds(l, LANES)]
        tz = tv_[r + 2, pl.ds(l, LANES)]
        vals = (ones, sx, sy, sz, tx, ty, tz,
                sx * tx, sx * ty, sx * tz,
                sy * tx, sy * ty, sy * tz,
                sz * tx, sz * ty, sz * tz)

        def fast_path():
          return (cur,) + tuple(a + v for a, v in zip(accs, vals))

        def slow_path():
          flush(cur, accs)
          notlast = ivec != last
          for j, v in enumerate(vals):
            plsc.addupdate_scatter(tab, [ivec + j * NSEG], v, mask=notlast)
          keep = jnp.where(notlast, 0.0, 1.0)
          return (last,) + tuple(v * keep for v in vals)

        return lax.cond((first == cur) & (last == cur), fast_path, slow_path)

      return lax.fori_loop(0, CHUNK // LANES, grp_body, carry)

    issue(0, sv0, tv0, iv0, semA)

    def pair_body(it, carry):
      sub0 = it * 2
      issue(sub0 + 1, sv1, tv1, iv1, semB)
      drain(sub0, sv0, tv0, iv0, semA)
      carry = process(sv0, tv0, iv0, carry)

      @pl.when(sub0 + 2 < nsub)
      def _():
        issue(sub0 + 2, sv0, tv0, iv0, semA)
      drain(sub0 + 1, sv1, tv1, iv1, semB)
      return process(sv1, tv1, iv1, carry)

    init = (jnp.int32(-1),) + (zeros16,) * 16
    fin = lax.fori_loop(0, nsub // 2, pair_body, init)
    flush(fin[0], fin[1:])
    pltpu.sync_copy(tab, out_h.at[wid])

  return k(srcx, tgtx, idx)


def _solve_call(partials):
  def body(p_ref, o_ref):
    s = jnp.sum(p_ref[...], axis=0)
    inv = 1.0 / jnp.maximum(s[0], 1.0)
    ss = (s[1], s[2], s[3])
    st = (s[4], s[5], s[6])
    ms = tuple(a * inv for a in ss)
    mt = tuple(a * inv for a in st)
    H = [[s[7 + 3 * a + b] - ss[a] * st[b] * inv for b in range(3)]
         for a in range(3)]
    (Sxx, Sxy, Sxz), (Syx, Syy, Syz), (Szx, Szy, Szz) = H
    n00 = Sxx + Syy + Szz
    n01 = Syz - Szy
    n02 = Szx - Sxz
    n03 = Sxy - Syx
    n11 = Sxx - Syy - Szz
    n12 = Sxy + Syx
    n13 = Szx + Sxz
    n22 = -Sxx + Syy - Szz
    n23 = Syz + Szy
    n33 = -Sxx - Syy + Szz
    fro2 = sum(H[a][b] * H[a][b] for a in range(3) for b in range(3))
    shift = jnp.sqrt(3.0 * fro2) + 1e-30
    B = [[n00 + shift, n01, n02, n03],
         [n01, n11 + shift, n12, n13],
         [n02, n12, n22 + shift, n23],
         [n03, n13, n23, n33 + shift]]
    for _ in range(16):
      C = [[sum(B[i][k] * B[k][j] for k in range(4)) for j in range(4)]
           for i in range(4)]
      invtr = 1.0 / jnp.maximum(C[0][0] + C[1][1] + C[2][2] + C[3][3], 1e-30)
      B = [[C[i][j] * invtr for j in range(4)] for i in range(4)]
    d = [B[i][i] for i in range(4)]
    m0 = (d[0] >= d[1]) & (d[0] >= d[2]) & (d[0] >= d[3])
    m1 = (d[1] >= d[2]) & (d[1] >= d[3])
    m2 = d[2] >= d[3]
    q = [jnp.where(m0, B[i][0],
         jnp.where(m1, B[i][1],
         jnp.where(m2, B[i][2], B[i][3]))) for i in range(4)]
    qn = 1.0 / jnp.sqrt(q[0] * q[0] + q[1] * q[1] + q[2] * q[2]
                        + q[3] * q[3] + 1e-30)
    w, x, y, z = (qi * qn for qi in q)
    r = [1.0 - 2.0 * (y * y + z * z), 2.0 * (x * y - w * z), 2.0 * (x * z + w * y),
         2.0 * (x * y + w * z), 1.0 - 2.0 * (x * x + z * z), 2.0 * (y * z - w * x),
         2.0 * (x * z - w * y), 2.0 * (y * z + w * x), 1.0 - 2.0 * (x * x + y * y)]
    t = [mt[a] - (r[3 * a] * ms[0] + r[3 * a + 1] * ms[1] + r[3 * a + 2] * ms[2])
         for a in range(3)]
    for j in range(9):
      o_ref[j, :] = r[j]
    for a in range(3):
      o_ref[9 + a, :] = t[a]
    zero = jnp.zeros((NSEG,), jnp.float32)
    for j in range(12, 16):
      o_ref[j, :] = zero

  return pl.pallas_call(
      body,
      out_shape=jax.ShapeDtypeStruct((16, NSEG), jnp.float32),
  )(partials)


def _apply_call(srcx, idx, rt_flat):
  n = idx.shape[0]
  ppt = n // NW
  nsub = ppt // CHUNK
  mesh = plsc.VectorSubcoreMesh(core_axis_name="c", subcore_axis_name="s")

  @functools.partial(
      pl.kernel, mesh=mesh,
      compiler_params=pltpu.CompilerParams(needs_layout_passes=False),
      out_type=jax.ShapeDtypeStruct((n // 32, 128), jnp.float32),
      scratch_types=[
          pltpu.VMEM((CROWS, 128), jnp.float32),
          pltpu.VMEM((CROWS, 128), jnp.float32),
          pltpu.VMEM((CHUNK,), jnp.int32),
          pltpu.VMEM((CHUNK,), jnp.int32),
          pltpu.VMEM((CROWS, 128), jnp.float32),
          pltpu.VMEM((CROWS, 128), jnp.float32),
          pltpu.VMEM((12 * NSEG,), jnp.float32),
          pltpu.SemaphoreType.DMA,
          pltpu.SemaphoreType.DMA,
          pltpu.SemaphoreType.DMA,
          pltpu.SemaphoreType.DMA,
      ],
  )
  def k(src_h, idx_h, rt_h, out_h, sv0, sv1, iv0, iv1, ov0, ov1,
        rtv, semA, semB, semC, semD):
    wid = lax.axis_index("s") * NC + lax.axis_index("c")
    pltpu.sync_copy(rt_h, rtv)
    zeros16 = jnp.zeros((LANES,), jnp.float32)

    def rb(sub):
      return pl.multiple_of((wid * ppt + sub * CHUNK) // 128 * 4, CROWS)

    def copies(sub, sv_, iv_, sem):
      ibase = wid * ppt + sub * CHUNK
      return (
          pltpu.make_async_copy(src_h.at[pl.ds(rb(sub), CROWS)], sv_, sem),
          pltpu.make_async_copy(idx_h.at[pl.ds(ibase, CHUNK)], iv_, sem),
      )

    def issue(sub, sv_, iv_, sem):
      for cpy in copies(sub, sv_, iv_, sem):
        cpy.start()

    def drain(sub, sv_, iv_, sem):
      for cpy in copies(sub, sv_, iv_, sem):
        cpy.wait()

    def out_copy(sub, ov_, sem):
      return pltpu.make_async_copy(ov_, out_h.at[pl.ds(rb(sub), CROWS)], sem)

    def process(sub, sv_, iv_, ov_, carry):

      def grp_body(g, c):
        r = g // 8 * 4
        l = g % 8 * LANES
        ivec = iv_[pl.ds(g * LANES, LANES)]
        sx = sv_[r, pl.ds(l, LANES)]
        sy = sv_[r + 1, pl.ds(l, LANES)]
        sz = sv_[r + 2, pl.ds(l, LANES)]
        co = tuple(
            plsc.load_gather(rtv, [ivec + j * NSEG]) for j in range(12))
        ov_[r, pl.ds(l, LANES)] = co[0] * sx + co[1] * sy + co[2] * sz + co[9]
        ov_[r + 1, pl.ds(l, LANES)] = (
            co[3] * sx + co[4] * sy + co[5] * sz + co[10])
        ov_[r + 2, pl.ds(l, LANES)] = (
            co[6] * sx + co[7] * sy + co[8] * sz + co[11])
        return c

      return lax.fori_loop(0, CHUNK // LANES, grp_body, carry)

    issue(0, sv0, iv0, semA)

    def pair_body(it, carry):
      sub0 = it * 2
      issue(sub0 + 1, sv1, iv1, semB)
      drain(sub0, sv0, iv0, semA)

      @pl.when(it > 0)
      def _():
        out_copy(sub0 - 2, ov0, semC).wait()
      carry = process(sub0, sv0, iv0, ov0, carry)
      out_copy(sub0, ov0, semC).start()

      @pl.when(sub0 + 2 < nsub)
      def _():
        issue(sub0 + 2, sv0, iv0, semA)
      drain(sub0 + 1, sv1, iv1, semB)

      @pl.when(it > 0)
      def _():
        out_copy(sub0 - 1, ov1, semD).wait()
      carry = process(sub0 + 1, sv1, iv1, ov1, carry)
      out_copy(sub0 + 1, ov1, semD).start()
      return carry

    lax.fori_loop(0, nsub // 2, pair_body, 0)
    out_copy(nsub - 2, ov0, semC).wait()
    out_copy(nsub - 1, ov1, semD).wait()

  return k(srcx, idx, rt_flat)


def kernel(src_points, tgt_points, batch_indices):
  n = src_points.shape[0]
  src = src_points.astype(jnp.float32)
  tgt = tgt_points.astype(jnp.float32)
  idx = batch_indices.astype(jnp.int32)
  srcx = _to_rows(src, n)
  tgtx = _to_rows(tgt, n)
  partials = _pass1_call(srcx, tgtx, idx)
  rt = _solve_call(partials.reshape(NW, 16, NSEG))
  alignedx = _apply_call(srcx, idx, rt[:12].reshape(12 * NSEG))
  aligned = alignedx.reshape(n // 128, 4, 128).swapaxes(1, 2).reshape(n, 4)[:, :3]
  R = jnp.transpose(rt[:9]).reshape(NSEG, 3, 3)
  t = jnp.transpose(rt[9:12])
  return (aligned, (R, t))


# speedup vs baseline: 31.0481x; 1.2058x over previous
import functools

import jax
import jax.numpy as jnp
from jax import lax
from jax.experimental import pallas as pl
from jax.experimental.pallas import tpu as pltpu
from jax.experimental.pallas import tpu_sc as plsc

NSEG = 2048
NC = 2
NS = 16
NW = NC * NS
LANES = 16
CHUNK = 1024
CROWS = CHUNK // 128 * 4
TABLE = 16 * NSEG


def _to_rows(pts, n):
  p = jnp.pad(pts, ((0, 0), (0, 1)))
  return p.reshape(n // 128, 128, 4).swapaxes(1, 2).reshape(n // 32, 128)


def _pass1_call(srcx, tgtx, idx):
  n = idx.shape[0]
  ppt = n // NW
  nsub = ppt // CHUNK
  mesh = plsc.VectorSubcoreMesh(core_axis_name="c", subcore_axis_name="s")

  @functools.partial(
      pl.kernel, mesh=mesh,
      compiler_params=pltpu.CompilerParams(needs_layout_passes=False),
      out_type=jax.ShapeDtypeStruct((NW, TABLE), jnp.float32),
      scratch_types=[
          pltpu.VMEM((CROWS, 128), jnp.float32),
          pltpu.VMEM((CROWS, 128), jnp.float32),
          pltpu.VMEM((CROWS, 128), jnp.float32),
          pltpu.VMEM((CROWS, 128), jnp.float32),
          pltpu.VMEM((CHUNK,), jnp.int32),
          pltpu.VMEM((CHUNK,), jnp.int32),
          pltpu.VMEM((TABLE,), jnp.float32),
          pltpu.SemaphoreType.DMA,
          pltpu.SemaphoreType.DMA,
      ],
  )
  def k(src_h, tgt_h, idx_h, out_h, sv0, tv0, sv1, tv1, iv0, iv1,
        tab, semA, semB):
    wid = lax.axis_index("s") * NC + lax.axis_index("c")
    ones = jnp.ones((LANES,), jnp.float32)
    zeros16 = jnp.zeros((LANES,), jnp.float32)
    iota = lax.iota(jnp.int32, LANES)
    iota_seg = iota * NSEG

    def zero_body(i, carry):
      for j in range(8):
        tab[pl.ds(i * (8 * LANES) + j * LANES, LANES)] = zeros16
      return carry
    lax.fori_loop(0, TABLE // (8 * LANES), zero_body, 0)

    def copies(sub, sv_, tv_, iv_, sem):
      rbase = pl.multiple_of((wid * ppt + sub * CHUNK) // 128 * 4, CROWS)
      ibase = wid * ppt + sub * CHUNK
      return (
          pltpu.make_async_copy(src_h.at[pl.ds(rbase, CROWS)], sv_, sem),
          pltpu.make_async_copy(tgt_h.at[pl.ds(rbase, CROWS)], tv_, sem),
          pltpu.make_async_copy(idx_h.at[pl.ds(ibase, CHUNK)], iv_, sem),
      )

    def issue(sub, sv_, tv_, iv_, sem):
      for cpy in copies(sub, sv_, tv_, iv_, sem):
        cpy.start()

    def drain(sub, sv_, tv_, iv_, sem):
      for cpy in copies(sub, sv_, tv_, iv_, sem):
        cpy.wait()

    def flush(cur, accs):
      @pl.when(cur >= 0)
      def _():
        row = zeros16
        for j in range(16):
          row = jnp.where(iota == j, jnp.sum(accs[j]), row)
        plsc.addupdate_scatter(tab, [iota_seg + cur], row)

    def process(sv_, tv_, iv_, carry):

      def moments(r, l):
        sx = sv_[r, pl.ds(l, LANES)]
        sy = sv_[r + 1, pl.ds(l, LANES)]
        sz = sv_[r + 2, pl.ds(l, LANES)]
        tx = tv_[r, pl.ds(l, LANES)]
        ty = tv_[r + 1, pl.ds(l, LANES)]
        tz = tv_[r + 2, pl.ds(l, LANES)]
        return (ones, sx, sy, sz, tx, ty, tz,
                sx * tx, sx * ty, sx * tz,
                sy * tx, sy * ty, sy * tz,
                sz * tx, sz * ty, sz * tz)

      def blk_body(b, c):
        cur = c[0]
        r = b * 4
        head = iv_[pl.ds(b * 128, LANES)]
        tail = iv_[pl.ds(b * 128 + 128 - LANES, LANES)]
        first = head[0]
        last = tail[LANES - 1]

        def fast_block():
          accs = list(c[1:])
          for gg in range(8):
            vals = moments(r, gg * LANES)
            accs = [a + v for a, v in zip(accs, vals)]
          return (cur,) + tuple(accs)

        def slow_block():
          def grp_body(gg, cc):
            gcur = cc[0]
            accs = cc[1:]
            l = gg * LANES
            ivec = iv_[pl.ds(b * 128 + l, LANES)]
            gfirst = ivec[0]
            glast = ivec[LANES - 1]
            vals = moments(r, l)

            def fast_path():
              return (gcur,) + tuple(a + v for a, v in zip(accs, vals))

            def slow_path():
              flush(gcur, accs)
              notlast = ivec != glast
              for j, v in enumerate(vals):
                plsc.addupdate_scatter(tab, [ivec + j * NSEG], v,
                                       mask=notlast)
              keep = jnp.where(notlast, 0.0, 1.0)
              return (glast,) + tuple(v * keep for v in vals)

            return lax.cond((gfirst == gcur) & (glast == gcur),
                            fast_path, slow_path)

          return lax.fori_loop(0, 8, grp_body, c)

        return lax.cond((first == cur) & (last == cur),
                        fast_block, slow_block)

      return lax.fori_loop(0, CHUNK // 128, blk_body, carry)

    issue(0, sv0, tv0, iv0, semA)

    def pair_body(it, carry):
      sub0 = it * 2
      issue(sub0 + 1, sv1, tv1, iv1, semB)
      drain(sub0, sv0, tv0, iv0, semA)
      carry = process(sv0, tv0, iv0, carry)

      @pl.when(sub0 + 2 < nsub)
      def _():
        issue(sub0 + 2, sv0, tv0, iv0, semA)
      drain(sub0 + 1, sv1, tv1, iv1, semB)
      return process(sv1, tv1, iv1, carry)

    init = (jnp.int32(-1),) + (zeros16,) * 16
    fin = lax.fori_loop(0, nsub // 2, pair_body, init)
    flush(fin[0], fin[1:])
    pltpu.sync_copy(tab, out_h.at[wid])

  return k(srcx, tgtx, idx)


def _solve_call(partials):
  def body(p_ref, o_ref):
    s = jnp.sum(p_ref[...], axis=0)
    inv = 1.0 / jnp.maximum(s[0], 1.0)
    ss = (s[1], s[2], s[3])
    st = (s[4], s[5], s[6])
    ms = tuple(a * inv for a in ss)
    mt = tuple(a * inv for a in st)
    H = [[s[7 + 3 * a + b] - ss[a] * st[b] * inv for b in range(3)]
         for a in range(3)]
    (Sxx, Sxy, Sxz), (Syx, Syy, Syz), (Szx, Szy, Szz) = H
    n00 = Sxx + Syy + Szz
    n01 = Syz - Szy
    n02 = Szx - Sxz
    n03 = Sxy - Syx
    n11 = Sxx - Syy - Szz
    n12 = Sxy + Syx
    n13 = Szx + Sxz
    n22 = -Sxx + Syy - Szz
    n23 = Syz + Szy
    n33 = -Sxx - Syy + Szz
    fro2 = sum(H[a][b] * H[a][b] for a in range(3) for b in range(3))
    shift = jnp.sqrt(3.0 * fro2) + 1e-30
    B = [[n00 + shift, n01, n02, n03],
         [n01, n11 + shift, n12, n13],
         [n02, n12, n22 + shift, n23],
         [n03, n13, n23, n33 + shift]]
    for _ in range(16):
      C = [[sum(B[i][k] * B[k][j] for k in range(4)) for j in range(4)]
           for i in range(4)]
      invtr = 1.0 / jnp.maximum(C[0][0] + C[1][1] + C[2][2] + C[3][3], 1e-30)
      B = [[C[i][j] * invtr for j in range(4)] for i in range(4)]
    d = [B[i][i] for i in range(4)]
    m0 = (d[0] >= d[1]) & (d[0] >= d[2]) & (d[0] >= d[3])
    m1 = (d[1] >= d[2]) & (d[1] >= d[3])
    m2 = d[2] >= d[3]
    q = [jnp.where(m0, B[i][0],
         jnp.where(m1, B[i][1],
         jnp.where(m2, B[i][2], B[i][3]))) for i in range(4)]
    qn = 1.0 / jnp.sqrt(q[0] * q[0] + q[1] * q[1] + q[2] * q[2]
                        + q[3] * q[3] + 1e-30)
    w, x, y, z = (qi * qn for qi in q)
    r = [1.0 - 2.0 * (y * y + z * z), 2.0 * (x * y - w * z), 2.0 * (x * z + w * y),
         2.0 * (x * y + w * z), 1.0 - 2.0 * (x * x + z * z), 2.0 * (y * z - w * x),
         2.0 * (x * z - w * y), 2.0 * (y * z + w * x), 1.0 - 2.0 * (x * x + y * y)]
    t = [mt[a] - (r[3 * a] * ms[0] + r[3 * a + 1] * ms[1] + r[3 * a + 2] * ms[2])
         for a in range(3)]
    for j in range(9):
      o_ref[j, :] = r[j]
    for a in range(3):
      o_ref[9 + a, :] = t[a]
    zero = jnp.zeros((NSEG,), jnp.float32)
    for j in range(12, 16):
      o_ref[j, :] = zero

  return pl.pallas_call(
      body,
      out_shape=jax.ShapeDtypeStruct((16, NSEG), jnp.float32),
  )(partials)


def _apply_call(srcx, idx, rt_flat):
  n = idx.shape[0]
  ppt = n // NW
  nsub = ppt // CHUNK
  mesh = plsc.VectorSubcoreMesh(core_axis_name="c", subcore_axis_name="s")

  @functools.partial(
      pl.kernel, mesh=mesh,
      compiler_params=pltpu.CompilerParams(needs_layout_passes=False),
      out_type=jax.ShapeDtypeStruct((n // 32, 128), jnp.float32),
      scratch_types=[
          pltpu.VMEM((CROWS, 128), jnp.float32),
          pltpu.VMEM((CROWS, 128), jnp.float32),
          pltpu.VMEM((CHUNK,), jnp.int32),
          pltpu.VMEM((CHUNK,), jnp.int32),
          pltpu.VMEM((CROWS, 128), jnp.float32),
          pltpu.VMEM((CROWS, 128), jnp.float32),
          pltpu.VMEM((12 * NSEG,), jnp.float32),
          pltpu.SemaphoreType.DMA,
          pltpu.SemaphoreType.DMA,
          pltpu.SemaphoreType.DMA,
          pltpu.SemaphoreType.DMA,
      ],
  )
  def k(src_h, idx_h, rt_h, out_h, sv0, sv1, iv0, iv1, ov0, ov1,
        rtv, semA, semB, semC, semD):
    wid = lax.axis_index("s") * NC + lax.axis_index("c")
    pltpu.sync_copy(rt_h, rtv)
    zeros16 = jnp.zeros((LANES,), jnp.float32)

    def rb(sub):
      return pl.multiple_of((wid * ppt + sub * CHUNK) // 128 * 4, CROWS)

    def copies(sub, sv_, iv_, sem):
      ibase = wid * ppt + sub * CHUNK
      return (
          pltpu.make_async_copy(src_h.at[pl.ds(rb(sub), CROWS)], sv_, sem),
          pltpu.make_async_copy(idx_h.at[pl.ds(ibase, CHUNK)], iv_, sem),
      )

    def issue(sub, sv_, iv_, sem):
      for cpy in copies(sub, sv_, iv_, sem):
        cpy.start()

    def drain(sub, sv_, iv_, sem):
      for cpy in copies(sub, sv_, iv_, sem):
        cpy.wait()

    def out_copy(sub, ov_, sem):
      return pltpu.make_async_copy(ov_, out_h.at[pl.ds(rb(sub), CROWS)], sem)

    def process(sub, sv_, iv_, ov_, carry):

      def emit(r, l, co):
        sx = sv_[r, pl.ds(l, LANES)]
        sy = sv_[r + 1, pl.ds(l, LANES)]
        sz = sv_[r + 2, pl.ds(l, LANES)]
        ov_[r, pl.ds(l, LANES)] = co[0] * sx + co[1] * sy + co[2] * sz + co[9]
        ov_[r + 1, pl.ds(l, LANES)] = (
            co[3] * sx + co[4] * sy + co[5] * sz + co[10])
        ov_[r + 2, pl.ds(l, LANES)] = (
            co[6] * sx + co[7] * sy + co[8] * sz + co[11])

      def blk_body(b, c):
        r = b * 4
        head = iv_[pl.ds(b * 128, LANES)]
        tail = iv_[pl.ds(b * 128 + 128 - LANES, LANES)]

        def fast_block():
          co = tuple(
              plsc.load_gather(rtv, [head + j * NSEG]) for j in range(12))
          for gg in range(8):
            emit(r, gg * LANES, co)
          return c

        def slow_block():
          def grp_body(gg, cc):
            ivec = iv_[pl.ds(b * 128 + gg * LANES, LANES)]
            co = tuple(
                plsc.load_gather(rtv, [ivec + j * NSEG]) for j in range(12))
            emit(r, gg * LANES, co)
            return cc
          return lax.fori_loop(0, 8, grp_body, c)

        return lax.cond(head[0] == tail[LANES - 1], fast_block, slow_block)

      return lax.fori_loop(0, CHUNK // 128, blk_body, carry)

    issue(0, sv0, iv0, semA)

    def pair_body(it, carry):
      sub0 = it * 2
      issue(sub0 + 1, sv1, iv1, semB)
      drain(sub0, sv0, iv0, semA)

      @pl.when(it > 0)
      def _():
        out_copy(sub0 - 2, ov0, semC).wait()
      carry = process(sub0, sv0, iv0, ov0, carry)
      out_copy(sub0, ov0, semC).start()

      @pl.when(sub0 + 2 < nsub)
      def _():
        issue(sub0 + 2, sv0, iv0, semA)
      drain(sub0 + 1, sv1, iv1, semB)

      @pl.when(it > 0)
      def _():
        out_copy(sub0 - 1, ov1, semD).wait()
      carry = process(sub0 + 1, sv1, iv1, ov1, carry)
      out_copy(sub0 + 1, ov1, semD).start()
      return carry

    lax.fori_loop(0, nsub // 2, pair_body, 0)
    out_copy(nsub - 2, ov0, semC).wait()
    out_copy(nsub - 1, ov1, semD).wait()

  return k(srcx, idx, rt_flat)


def kernel(src_points, tgt_points, batch_indices):
  n = src_points.shape[0]
  src = src_points.astype(jnp.float32)
  tgt = tgt_points.astype(jnp.float32)
  idx = batch_indices.astype(jnp.int32)
  srcx = _to_rows(src, n)
  tgtx = _to_rows(tgt, n)
  partials = _pass1_call(srcx, tgtx, idx)
  rt = _solve_call(partials.reshape(NW, 16, NSEG))
  alignedx = _apply_call(srcx, idx, rt[:12].reshape(12 * NSEG))
  aligned = alignedx.reshape(n // 128, 4, 128).swapaxes(1, 2).reshape(n, 4)[:, :3]
  R = jnp.transpose(rt[:9]).reshape(NSEG, 3, 3)
  t = jnp.transpose(rt[9:12])
  return (aligned, (R, t))


# speedup vs baseline: 31.8585x; 1.0261x over previous
import functools

import jax
import jax.numpy as jnp
from jax import lax
from jax.experimental import pallas as pl
from jax.experimental.pallas import tpu as pltpu
from jax.experimental.pallas import tpu_sc as plsc

NSEG = 2048
NC = 2
NS = 16
NW = NC * NS
LANES = 16
CHUNK = 2048
CROWS = CHUNK // 128 * 4
TABLE = 16 * NSEG


def _to_rows(pts, n):
  p = jnp.pad(pts, ((0, 0), (0, 1)))
  return p.reshape(n // 128, 128, 4).swapaxes(1, 2).reshape(n // 32, 128)


def _pass1_call(srcx, tgtx, idx):
  n = idx.shape[0]
  ppt = n // NW
  nsub = ppt // CHUNK
  mesh = plsc.VectorSubcoreMesh(core_axis_name="c", subcore_axis_name="s")

  @functools.partial(
      pl.kernel, mesh=mesh,
      compiler_params=pltpu.CompilerParams(needs_layout_passes=False),
      out_type=jax.ShapeDtypeStruct((NW, TABLE), jnp.float32),
      scratch_types=[
          pltpu.VMEM((CROWS, 128), jnp.float32),
          pltpu.VMEM((CROWS, 128), jnp.float32),
          pltpu.VMEM((CROWS, 128), jnp.float32),
          pltpu.VMEM((CROWS, 128), jnp.float32),
          pltpu.VMEM((CHUNK,), jnp.int32),
          pltpu.VMEM((CHUNK,), jnp.int32),
          pltpu.VMEM((TABLE,), jnp.float32),
          pltpu.SemaphoreType.DMA,
          pltpu.SemaphoreType.DMA,
      ],
  )
  def k(src_h, tgt_h, idx_h, out_h, sv0, tv0, sv1, tv1, iv0, iv1,
        tab, semA, semB):
    wid = lax.axis_index("s") * NC + lax.axis_index("c")
    ones = jnp.ones((LANES,), jnp.float32)
    zeros16 = jnp.zeros((LANES,), jnp.float32)
    iota = lax.iota(jnp.int32, LANES)
    iota_seg = iota * NSEG

    def zero_body(i, carry):
      for j in range(8):
        tab[pl.ds(i * (8 * LANES) + j * LANES, LANES)] = zeros16
      return carry
    lax.fori_loop(0, TABLE // (8 * LANES), zero_body, 0)

    def copies(sub, sv_, tv_, iv_, sem):
      rbase = pl.multiple_of((wid * ppt + sub * CHUNK) // 128 * 4, CROWS)
      ibase = wid * ppt + sub * CHUNK
      return (
          pltpu.make_async_copy(src_h.at[pl.ds(rbase, CROWS)], sv_, sem),
          pltpu.make_async_copy(tgt_h.at[pl.ds(rbase, CROWS)], tv_, sem),
          pltpu.make_async_copy(idx_h.at[pl.ds(ibase, CHUNK)], iv_, sem),
      )

    def issue(sub, sv_, tv_, iv_, sem):
      for cpy in copies(sub, sv_, tv_, iv_, sem):
        cpy.start()

    def drain(sub, sv_, tv_, iv_, sem):
      for cpy in copies(sub, sv_, tv_, iv_, sem):
        cpy.wait()

    def flush(cur, accs):
      @pl.when(cur >= 0)
      def _():
        row = zeros16
        for j in range(16):
          row = jnp.where(iota == j, jnp.sum(accs[j]), row)
        plsc.addupdate_scatter(tab, [iota_seg + cur], row)

    def process(sv_, tv_, iv_, carry):

      def moments(r, l):
        sx = sv_[r, pl.ds(l, LANES)]
        sy = sv_[r + 1, pl.ds(l, LANES)]
        sz = sv_[r + 2, pl.ds(l, LANES)]
        tx = tv_[r, pl.ds(l, LANES)]
        ty = tv_[r + 1, pl.ds(l, LANES)]
        tz = tv_[r + 2, pl.ds(l, LANES)]
        return (ones, sx, sy, sz, tx, ty, tz,
                sx * tx, sx * ty, sx * tz,
                sy * tx, sy * ty, sy * tz,
                sz * tx, sz * ty, sz * tz)

      def blk_body(b, c):
        cur = c[0]
        r = b * 4
        head = iv_[pl.ds(b * 128, LANES)]
        tail = iv_[pl.ds(b * 128 + 128 - LANES, LANES)]
        first = head[0]
        last = tail[LANES - 1]

        def fast_block():
          accs = list(c[1:])
          for gg in range(8):
            vals = moments(r, gg * LANES)
            accs = [a + v for a, v in zip(accs, vals)]
          return (cur,) + tuple(accs)

        def slow_block():
          def grp_body(gg, cc):
            gcur = cc[0]
            accs = cc[1:]
            l = gg * LANES
            ivec = iv_[pl.ds(b * 128 + l, LANES)]
            gfirst = ivec[0]
            glast = ivec[LANES - 1]
            vals = moments(r, l)

            def fast_path():
              return (gcur,) + tuple(a + v for a, v in zip(accs, vals))

            def slow_path():
              flush(gcur, accs)
              notlast = ivec != glast
              for j, v in enumerate(vals):
                plsc.addupdate_scatter(tab, [ivec + j * NSEG], v,
                                       mask=notlast)
              keep = jnp.where(notlast, 0.0, 1.0)
              return (glast,) + tuple(v * keep for v in vals)

            return lax.cond((gfirst == gcur) & (glast == gcur),
                            fast_path, slow_path)

          return lax.fori_loop(0, 8, grp_body, c)

        return lax.cond((first == cur) & (last == cur),
                        fast_block, slow_block)

      return lax.fori_loop(0, CHUNK // 128, blk_body, carry)

    issue(0, sv0, tv0, iv0, semA)

    def pair_body(it, carry):
      sub0 = it * 2
      issue(sub0 + 1, sv1, tv1, iv1, semB)
      drain(sub0, sv0, tv0, iv0, semA)
      carry = process(sv0, tv0, iv0, carry)

      @pl.when(sub0 + 2 < nsub)
      def _():
        issue(sub0 + 2, sv0, tv0, iv0, semA)
      drain(sub0 + 1, sv1, tv1, iv1, semB)
      return process(sv1, tv1, iv1, carry)

    init = (jnp.int32(-1),) + (zeros16,) * 16
    fin = lax.fori_loop(0, nsub // 2, pair_body, init)
    flush(fin[0], fin[1:])
    pltpu.sync_copy(tab, out_h.at[wid])

  return k(srcx, tgtx, idx)


def _solve_call(partials):
  def body(p_ref, o_ref):
    s = jnp.sum(p_ref[...], axis=0)
    inv = 1.0 / jnp.maximum(s[0], 1.0)
    ss = (s[1], s[2], s[3])
    st = (s[4], s[5], s[6])
    ms = tuple(a * inv for a in ss)
    mt = tuple(a * inv for a in st)
    H = [[s[7 + 3 * a + b] - ss[a] * st[b] * inv for b in range(3)]
         for a in range(3)]
    (Sxx, Sxy, Sxz), (Syx, Syy, Syz), (Szx, Szy, Szz) = H
    n00 = Sxx + Syy + Szz
    n01 = Syz - Szy
    n02 = Szx - Sxz
    n03 = Sxy - Syx
    n11 = Sxx - Syy - Szz
    n12 = Sxy + Syx
    n13 = Szx + Sxz
    n22 = -Sxx + Syy - Szz
    n23 = Syz + Szy
    n33 = -Sxx - Syy + Szz
    fro2 = sum(H[a][b] * H[a][b] for a in range(3) for b in range(3))
    shift = jnp.sqrt(3.0 * fro2) + 1e-30
    B = [[n00 + shift, n01, n02, n03],
         [n01, n11 + shift, n12, n13],
         [n02, n12, n22 + shift, n23],
         [n03, n13, n23, n33 + shift]]
    for _ in range(16):
      C = [[sum(B[i][k] * B[k][j] for k in range(4)) for j in range(4)]
           for i in range(4)]
      invtr = 1.0 / jnp.maximum(C[0][0] + C[1][1] + C[2][2] + C[3][3], 1e-30)
      B = [[C[i][j] * invtr for j in range(4)] for i in range(4)]
    d = [B[i][i] for i in range(4)]
    m0 = (d[0] >= d[1]) & (d[0] >= d[2]) & (d[0] >= d[3])
    m1 = (d[1] >= d[2]) & (d[1] >= d[3])
    m2 = d[2] >= d[3]
    q = [jnp.where(m0, B[i][0],
         jnp.where(m1, B[i][1],
         jnp.where(m2, B[i][2], B[i][3]))) for i in range(4)]
    qn = 1.0 / jnp.sqrt(q[0] * q[0] + q[1] * q[1] + q[2] * q[2]
                        + q[3] * q[3] + 1e-30)
    w, x, y, z = (qi * qn for qi in q)
    r = [1.0 - 2.0 * (y * y + z * z), 2.0 * (x * y - w * z), 2.0 * (x * z + w * y),
         2.0 * (x * y + w * z), 1.0 - 2.0 * (x * x + z * z), 2.0 * (y * z - w * x),
         2.0 * (x * z - w * y), 2.0 * (y * z + w * x), 1.0 - 2.0 * (x * x + y * y)]
    t = [mt[a] - (r[3 * a] * ms[0] + r[3 * a + 1] * ms[1] + r[3 * a + 2] * ms[2])
         for a in range(3)]
    for j in range(9):
      o_ref[j, :] = r[j]
    for a in range(3):
      o_ref[9 + a, :] = t[a]
    zero = jnp.zeros((NSEG,), jnp.float32)
    for j in range(12, 16):
      o_ref[j, :] = zero

  return pl.pallas_call(
      body,
      out_shape=jax.ShapeDtypeStruct((16, NSEG), jnp.float32),
  )(partials)


def _apply_call(srcx, idx, rt_flat):
  n = idx.shape[0]
  ppt = n // NW
  nsub = ppt // CHUNK
  mesh = plsc.VectorSubcoreMesh(core_axis_name="c", subcore_axis_name="s")

  @functools.partial(
      pl.kernel, mesh=mesh,
      compiler_params=pltpu.CompilerParams(needs_layout_passes=False),
      out_type=jax.ShapeDtypeStruct((n // 32, 128), jnp.float32),
      scratch_types=[
          pltpu.VMEM((CROWS, 128), jnp.float32),
          pltpu.VMEM((CROWS, 128), jnp.float32),
          pltpu.VMEM((CHUNK,), jnp.int32),
          pltpu.VMEM((CHUNK,), jnp.int32),
          pltpu.VMEM((CROWS, 128), jnp.float32),
          pltpu.VMEM((CROWS, 128), jnp.float32),
          pltpu.VMEM((12 * NSEG,), jnp.float32),
          pltpu.SemaphoreType.DMA,
          pltpu.SemaphoreType.DMA,
          pltpu.SemaphoreType.DMA,
          pltpu.SemaphoreType.DMA,
      ],
  )
  def k(src_h, idx_h, rt_h, out_h, sv0, sv1, iv0, iv1, ov0, ov1,
        rtv, semA, semB, semC, semD):
    wid = lax.axis_index("s") * NC + lax.axis_index("c")
    pltpu.sync_copy(rt_h, rtv)
    zeros16 = jnp.zeros((LANES,), jnp.float32)

    def rb(sub):
      return pl.multiple_of((wid * ppt + sub * CHUNK) // 128 * 4, CROWS)

    def copies(sub, sv_, iv_, sem):
      ibase = wid * ppt + sub * CHUNK
      return (
          pltpu.make_async_copy(src_h.at[pl.ds(rb(sub), CROWS)], sv_, sem),
          pltpu.make_async_copy(idx_h.at[pl.ds(ibase, CHUNK)], iv_, sem),
      )

    def issue(sub, sv_, iv_, sem):
      for cpy in copies(sub, sv_, iv_, sem):
        cpy.start()

    def drain(sub, sv_, iv_, sem):
      for cpy in copies(sub, sv_, iv_, sem):
        cpy.wait()

    def out_copy(sub, ov_, sem):
      return pltpu.make_async_copy(ov_, out_h.at[pl.ds(rb(sub), CROWS)], sem)

    def process(sub, sv_, iv_, ov_, carry):

      def emit(r, l, co):
        sx = sv_[r, pl.ds(l, LANES)]
        sy = sv_[r + 1, pl.ds(l, LANES)]
        sz = sv_[r + 2, pl.ds(l, LANES)]
        ov_[r, pl.ds(l, LANES)] = co[0] * sx + co[1] * sy + co[2] * sz + co[9]
        ov_[r + 1, pl.ds(l, LANES)] = (
            co[3] * sx + co[4] * sy + co[5] * sz + co[10])
        ov_[r + 2, pl.ds(l, LANES)] = (
            co[6] * sx + co[7] * sy + co[8] * sz + co[11])

      def blk_body(b, c):
        r = b * 4
        head = iv_[pl.ds(b * 128, LANES)]
        tail = iv_[pl.ds(b * 128 + 128 - LANES, LANES)]

        def fast_block():
          co = tuple(
              plsc.load_gather(rtv, [head + j * NSEG]) for j in range(12))
          for gg in range(8):
            emit(r, gg * LANES, co)
          return c

        def slow_block():
          def grp_body(gg, cc):
            ivec = iv_[pl.ds(b * 128 + gg * LANES, LANES)]
            co = tuple(
                plsc.load_gather(rtv, [ivec + j * NSEG]) for j in range(12))
            emit(r, gg * LANES, co)
            return cc
          return lax.fori_loop(0, 8, grp_body, c)

        return lax.cond(head[0] == tail[LANES - 1], fast_block, slow_block)

      return lax.fori_loop(0, CHUNK // 128, blk_body, carry)

    issue(0, sv0, iv0, semA)

    def pair_body(it, carry):
      sub0 = it * 2
      issue(sub0 + 1, sv1, iv1, semB)
      drain(sub0, sv0, iv0, semA)

      @pl.when(it > 0)
      def _():
        out_copy(sub0 - 2, ov0, semC).wait()
      carry = process(sub0, sv0, iv0, ov0, carry)
      out_copy(sub0, ov0, semC).start()

      @pl.when(sub0 + 2 < nsub)
      def _():
        issue(sub0 + 2, sv0, iv0, semA)
      drain(sub0 + 1, sv1, iv1, semB)

      @pl.when(it > 0)
      def _():
        out_copy(sub0 - 1, ov1, semD).wait()
      carry = process(sub0 + 1, sv1, iv1, ov1, carry)
      out_copy(sub0 + 1, ov1, semD).start()
      return carry

    lax.fori_loop(0, nsub // 2, pair_body, 0)
    out_copy(nsub - 2, ov0, semC).wait()
    out_copy(nsub - 1, ov1, semD).wait()

  return k(srcx, idx, rt_flat)


def kernel(src_points, tgt_points, batch_indices):
  n = src_points.shape[0]
  src = src_points.astype(jnp.float32)
  tgt = tgt_points.astype(jnp.float32)
  idx = batch_indices.astype(jnp.int32)
  srcx = _to_rows(src, n)
  tgtx = _to_rows(tgt, n)
  partials = _pass1_call(srcx, tgtx, idx)
  rt = _solve_call(partials.reshape(NW, 16, NSEG))
  alignedx = _apply_call(srcx, idx, rt[:12].reshape(12 * NSEG))
  aligned = alignedx.reshape(n // 128, 4, 128).swapaxes(1, 2).reshape(n, 4)[:, :3]
  R = jnp.transpose(rt[:9]).reshape(NSEG, 3, 3)
  t = jnp.transpose(rt[9:12])
  return (aligned, (R, t))


# speedup vs baseline: 32.1629x; 1.0096x over previous
import functools

import jax
import jax.numpy as jnp
from jax import lax
from jax.experimental import pallas as pl
from jax.experimental.pallas import tpu as pltpu
from jax.experimental.pallas import tpu_sc as plsc

NSEG = 2048
NC = 2
NS = 16
NW = NC * NS
LANES = 16
CHUNK = 2048
CROWS = CHUNK // 128 * 4
TABLE = 16 * NSEG


def _to_rows(pts, n):
  p = jnp.pad(pts, ((0, 0), (0, 1)))
  return p.reshape(n // 128, 128, 4).swapaxes(1, 2).reshape(n // 32, 128)


def _pass1_call(srcx, tgtx, idx):
  n = idx.shape[0]
  ppt = n // NW
  nsub = ppt // CHUNK
  mesh = plsc.VectorSubcoreMesh(core_axis_name="c", subcore_axis_name="s")

  @functools.partial(
      pl.kernel, mesh=mesh,
      compiler_params=pltpu.CompilerParams(needs_layout_passes=False),
      out_type=jax.ShapeDtypeStruct((NW, TABLE), jnp.float32),
      scratch_types=[
          pltpu.VMEM((CROWS, 128), jnp.float32),
          pltpu.VMEM((CROWS, 128), jnp.float32),
          pltpu.VMEM((CROWS, 128), jnp.float32),
          pltpu.VMEM((CROWS, 128), jnp.float32),
          pltpu.VMEM((CHUNK,), jnp.int32),
          pltpu.VMEM((CHUNK,), jnp.int32),
          pltpu.VMEM((TABLE,), jnp.float32),
          pltpu.SemaphoreType.DMA,
          pltpu.SemaphoreType.DMA,
      ],
  )
  def k(src_h, tgt_h, idx_h, out_h, sv0, tv0, sv1, tv1, iv0, iv1,
        tab, semA, semB):
    wid = lax.axis_index("s") * NC + lax.axis_index("c")
    ones = jnp.ones((LANES,), jnp.float32)
    zeros16 = jnp.zeros((LANES,), jnp.float32)
    iota = lax.iota(jnp.int32, LANES)
    iota_seg = iota * NSEG

    def zero_body(i, carry):
      for j in range(8):
        tab[pl.ds(i * (8 * LANES) + j * LANES, LANES)] = zeros16
      return carry
    lax.fori_loop(0, TABLE // (8 * LANES), zero_body, 0)

    def copies(sub, sv_, tv_, iv_, sem):
      rbase = pl.multiple_of((wid * ppt + sub * CHUNK) // 128 * 4, CROWS)
      ibase = wid * ppt + sub * CHUNK
      return (
          pltpu.make_async_copy(src_h.at[pl.ds(rbase, CROWS)], sv_, sem),
          pltpu.make_async_copy(tgt_h.at[pl.ds(rbase, CROWS)], tv_, sem),
          pltpu.make_async_copy(idx_h.at[pl.ds(ibase, CHUNK)], iv_, sem),
      )

    def issue(sub, sv_, tv_, iv_, sem):
      for cpy in copies(sub, sv_, tv_, iv_, sem):
        cpy.start()

    def drain(sub, sv_, tv_, iv_, sem):
      for cpy in copies(sub, sv_, tv_, iv_, sem):
        cpy.wait()

    def flush(cur, accs):
      @pl.when(cur >= 0)
      def _():
        row = zeros16
        for j in range(16):
          row = jnp.where(iota == j, jnp.sum(accs[j]), row)
        plsc.addupdate_scatter(tab, [iota_seg + cur], row)

    def process(sv_, tv_, iv_, carry):

      def moments(r, l):
        sx = sv_[r, pl.ds(l, LANES)]
        sy = sv_[r + 1, pl.ds(l, LANES)]
        sz = sv_[r + 2, pl.ds(l, LANES)]
        tx = tv_[r, pl.ds(l, LANES)]
        ty = tv_[r + 1, pl.ds(l, LANES)]
        tz = tv_[r + 2, pl.ds(l, LANES)]
        return (ones, sx, sy, sz, tx, ty, tz,
                sx * tx, sx * ty, sx * tz,
                sy * tx, sy * ty, sy * tz,
                sz * tx, sz * ty, sz * tz)

      def blk_body(b, c):
        cur = c[0]
        r = b * 4
        head = iv_[pl.ds(b * 128, LANES)]
        tail = iv_[pl.ds(b * 128 + 128 - LANES, LANES)]
        first = head[0]
        last = tail[LANES - 1]

        def fast_block():
          accs = list(c[1:])
          for gg in range(8):
            vals = moments(r, gg * LANES)
            accs = [a + v for a, v in zip(accs, vals)]
          return (cur,) + tuple(accs)

        def slow_block():
          def grp_body(gg, cc):
            gcur = cc[0]
            accs = cc[1:]
            l = gg * LANES
            ivec = iv_[pl.ds(b * 128 + l, LANES)]
            gfirst = ivec[0]
            glast = ivec[LANES - 1]
            vals = moments(r, l)

            def fast_path():
              return (gcur,) + tuple(a + v for a, v in zip(accs, vals))

            def slow_path():
              flush(gcur, accs)
              notlast = ivec != glast
              for j, v in enumerate(vals):
                plsc.addupdate_scatter(tab, [ivec + j * NSEG], v,
                                       mask=notlast)
              keep = jnp.where(notlast, 0.0, 1.0)
              return (glast,) + tuple(v * keep for v in vals)

            return lax.cond((gfirst == gcur) & (glast == gcur),
                            fast_path, slow_path)

          return lax.fori_loop(0, 8, grp_body, c)

        return lax.cond((first == cur) & (last == cur),
                        fast_block, slow_block)

      return lax.fori_loop(0, CHUNK // 128, blk_body, carry)

    issue(0, sv0, tv0, iv0, semA)

    def pair_body(it, carry):
      sub0 = it * 2
      issue(sub0 + 1, sv1, tv1, iv1, semB)
      drain(sub0, sv0, tv0, iv0, semA)
      carry = process(sv0, tv0, iv0, carry)

      @pl.when(sub0 + 2 < nsub)
      def _():
        issue(sub0 + 2, sv0, tv0, iv0, semA)
      drain(sub0 + 1, sv1, tv1, iv1, semB)
      return process(sv1, tv1, iv1, carry)

    init = (jnp.int32(-1),) + (zeros16,) * 16
    fin = lax.fori_loop(0, nsub // 2, pair_body, init)
    flush(fin[0], fin[1:])
    pltpu.sync_copy(tab, out_h.at[wid])

  return k(srcx, tgtx, idx)


def _solve_call(partials):
  def body(p_ref, o_ref):
    s = jnp.sum(p_ref[...], axis=0)
    inv = 1.0 / jnp.maximum(s[0], 1.0)
    ss = (s[1], s[2], s[3])
    st = (s[4], s[5], s[6])
    ms = tuple(a * inv for a in ss)
    mt = tuple(a * inv for a in st)
    H = [[s[7 + 3 * a + b] - ss[a] * st[b] * inv for b in range(3)]
         for a in range(3)]
    (Sxx, Sxy, Sxz), (Syx, Syy, Syz), (Szx, Szy, Szz) = H
    n00 = Sxx + Syy + Szz
    n01 = Syz - Szy
    n02 = Szx - Sxz
    n03 = Sxy - Syx
    n11 = Sxx - Syy - Szz
    n12 = Sxy + Syx
    n13 = Szx + Sxz
    n22 = -Sxx + Syy - Szz
    n23 = Syz + Szy
    n33 = -Sxx - Syy + Szz
    fro2 = sum(H[a][b] * H[a][b] for a in range(3) for b in range(3))
    shift = jnp.sqrt(3.0 * fro2) + 1e-30
    B = [[n00 + shift, n01, n02, n03],
         [n01, n11 + shift, n12, n13],
         [n02, n12, n22 + shift, n23],
         [n03, n13, n23, n33 + shift]]
    for _ in range(16):
      C = [[sum(B[i][k] * B[k][j] for k in range(4)) for j in range(4)]
           for i in range(4)]
      invtr = 1.0 / jnp.maximum(C[0][0] + C[1][1] + C[2][2] + C[3][3], 1e-30)
      B = [[C[i][j] * invtr for j in range(4)] for i in range(4)]
    d = [B[i][i] for i in range(4)]
    m0 = (d[0] >= d[1]) & (d[0] >= d[2]) & (d[0] >= d[3])
    m1 = (d[1] >= d[2]) & (d[1] >= d[3])
    m2 = d[2] >= d[3]
    q = [jnp.where(m0, B[i][0],
         jnp.where(m1, B[i][1],
         jnp.where(m2, B[i][2], B[i][3]))) for i in range(4)]
    qn = 1.0 / jnp.sqrt(q[0] * q[0] + q[1] * q[1] + q[2] * q[2]
                        + q[3] * q[3] + 1e-30)
    w, x, y, z = (qi * qn for qi in q)
    r = [1.0 - 2.0 * (y * y + z * z), 2.0 * (x * y - w * z), 2.0 * (x * z + w * y),
         2.0 * (x * y + w * z), 1.0 - 2.0 * (x * x + z * z), 2.0 * (y * z - w * x),
         2.0 * (x * z - w * y), 2.0 * (y * z + w * x), 1.0 - 2.0 * (x * x + y * y)]
    t = [mt[a] - (r[3 * a] * ms[0] + r[3 * a + 1] * ms[1] + r[3 * a + 2] * ms[2])
         for a in range(3)]
    for j in range(9):
      o_ref[j, :] = r[j]
    for a in range(3):
      o_ref[9 + a, :] = t[a]
    zero = jnp.zeros((NSEG,), jnp.float32)
    for j in range(12, 16):
      o_ref[j, :] = zero

  return pl.pallas_call(
      body,
      out_shape=jax.ShapeDtypeStruct((16, NSEG), jnp.float32),
  )(partials)


def _apply_call(srcx, idx, rt_flat):
  n = idx.shape[0]
  ppt = n // NW
  nsub = ppt // CHUNK
  mesh = plsc.VectorSubcoreMesh(core_axis_name="c", subcore_axis_name="s")

  @functools.partial(
      pl.kernel, mesh=mesh,
      compiler_params=pltpu.CompilerParams(needs_layout_passes=False),
      out_type=jax.ShapeDtypeStruct((n // 32, 128), jnp.float32),
      scratch_types=[
          pltpu.VMEM((CROWS, 128), jnp.float32),
          pltpu.VMEM((CROWS, 128), jnp.float32),
          pltpu.VMEM((CHUNK,), jnp.int32),
          pltpu.VMEM((CHUNK,), jnp.int32),
          pltpu.VMEM((CROWS, 128), jnp.float32),
          pltpu.VMEM((CROWS, 128), jnp.float32),
          pltpu.VMEM((12 * NSEG,), jnp.float32),
          pltpu.SemaphoreType.DMA,
          pltpu.SemaphoreType.DMA,
          pltpu.SemaphoreType.DMA,
          pltpu.SemaphoreType.DMA,
      ],
  )
  def k(src_h, idx_h, rt_h, out_h, sv0, sv1, iv0, iv1, ov0, ov1,
        rtv, semA, semB, semC, semD):
    wid = lax.axis_index("s") * NC + lax.axis_index("c")
    pltpu.sync_copy(rt_h, rtv)
    zeros16 = jnp.zeros((LANES,), jnp.float32)

    def rb(sub):
      return pl.multiple_of((wid * ppt + sub * CHUNK) // 128 * 4, CROWS)

    def copies(sub, sv_, iv_, sem):
      ibase = wid * ppt + sub * CHUNK
      return (
          pltpu.make_async_copy(src_h.at[pl.ds(rb(sub), CROWS)], sv_, sem),
          pltpu.make_async_copy(idx_h.at[pl.ds(ibase, CHUNK)], iv_, sem),
      )

    def issue(sub, sv_, iv_, sem):
      for cpy in copies(sub, sv_, iv_, sem):
        cpy.start()

    def drain(sub, sv_, iv_, sem):
      for cpy in copies(sub, sv_, iv_, sem):
        cpy.wait()

    def out_copy(sub, ov_, sem):
      return pltpu.make_async_copy(ov_, out_h.at[pl.ds(rb(sub), CROWS)], sem)

    def process(sub, sv_, iv_, ov_, carry):

      def emit(r, l, co):
        sx = sv_[r, pl.ds(l, LANES)]
        sy = sv_[r + 1, pl.ds(l, LANES)]
        sz = sv_[r + 2, pl.ds(l, LANES)]
        ov_[r, pl.ds(l, LANES)] = co[0] * sx + co[1] * sy + co[2] * sz + co[9]
        ov_[r + 1, pl.ds(l, LANES)] = (
            co[3] * sx + co[4] * sy + co[5] * sz + co[10])
        ov_[r + 2, pl.ds(l, LANES)] = (
            co[6] * sx + co[7] * sy + co[8] * sz + co[11])

      @plsc.parallel_loop(0, CHUNK // 128, unroll=2)
      def blk_body(b):
        r = b * 4
        head = iv_[pl.ds(b * 128, LANES)]
        tail = iv_[pl.ds(b * 128 + 128 - LANES, LANES)]

        def fast_block():
          co = tuple(
              plsc.load_gather(rtv, [head + j * NSEG]) for j in range(12))
          for gg in range(8):
            emit(r, gg * LANES, co)

        def slow_block():
          for gg in range(8):
            ivec = iv_[pl.ds(b * 128 + gg * LANES, LANES)]
            co = tuple(
                plsc.load_gather(rtv, [ivec + j * NSEG]) for j in range(12))
            emit(r, gg * LANES, co)

        lax.cond(head[0] == tail[LANES - 1], fast_block, slow_block)

      return carry

    issue(0, sv0, iv0, semA)

    def pair_body(it, carry):
      sub0 = it * 2
      issue(sub0 + 1, sv1, iv1, semB)
      drain(sub0, sv0, iv0, semA)

      @pl.when(it > 0)
      def _():
        out_copy(sub0 - 2, ov0, semC).wait()
      carry = process(sub0, sv0, iv0, ov0, carry)
      out_copy(sub0, ov0, semC).start()

      @pl.when(sub0 + 2 < nsub)
      def _():
        issue(sub0 + 2, sv0, iv0, semA)
      drain(sub0 + 1, sv1, iv1, semB)

      @pl.when(it > 0)
      def _():
        out_copy(sub0 - 1, ov1, semD).wait()
      carry = process(sub0 + 1, sv1, iv1, ov1, carry)
      out_copy(sub0 + 1, ov1, semD).start()
      return carry

    lax.fori_loop(0, nsub // 2, pair_body, 0)
    out_copy(nsub - 2, ov0, semC).wait()
    out_copy(nsub - 1, ov1, semD).wait()

  return k(srcx, idx, rt_flat)


def kernel(src_points, tgt_points, batch_indices):
  n = src_points.shape[0]
  src = src_points.astype(jnp.float32)
  tgt = tgt_points.astype(jnp.float32)
  idx = batch_indices.astype(jnp.int32)
  srcx = _to_rows(src, n)
  tgtx = _to_rows(tgt, n)
  partials = _pass1_call(srcx, tgtx, idx)
  rt = _solve_call(partials.reshape(NW, 16, NSEG))
  alignedx = _apply_call(srcx, idx, rt[:12].reshape(12 * NSEG))
  aligned = alignedx.reshape(n // 128, 4, 128).swapaxes(1, 2).reshape(n, 4)[:, :3]
  R = jnp.transpose(rt[:9]).reshape(NSEG, 3, 3)
  t = jnp.transpose(rt[9:12])
  return (aligned, (R, t))
